# Initial kernel scaffold; baseline (speedup 1.0000x reference)
import jax
import jax.numpy as jnp
from jax.experimental import pallas as pl


def kernel(x_i, x_j, edge_index_i, edge_index_j, W1, b1, W2, b2, W3, b3, W_att, W_ntn, V_ntn, b_ntn, W_m0, b_m0, W_m1, b_m1, W_m2, b_m2, W_m3, b_m3, W_s, b_s):
    raise NotImplementedError("write your pallas kernel here")


# speedup vs baseline: 12.8545x; 12.8545x over previous
import functools

import jax
import jax.numpy as jnp
from jax import lax
from jax.experimental import pallas as pl
from jax.experimental.pallas import tpu as pltpu
from jax.experimental.pallas import tpu_sc as plsc

N = 10000
NP = 10112
E = 320000
D_IN = 128
F1, F2, F3 = 64, 32, 16
NTN_SLICES = 16
HIST_BINS = 16

NWORK = 32
CH = 128
KCH = 79
EP = NWORK * KCH * CH
STRIPE = NP // 16


def _make_degree_kernel():
    DW = 16
    mesh = plsc.VectorSubcoreMesh(core_axis_name="c", subcore_axis_name="s")

    @functools.partial(
        pl.kernel,
        out_type=jax.ShapeDtypeStruct((4 * NP, DW), jnp.float32),
        mesh=mesh,
        compiler_params=pltpu.CompilerParams(use_tc_tiling_on_sc=False),
        scratch_types=[
            pltpu.VMEM_SHARED((NP, DW), jnp.float32),
            pltpu.VMEM_SHARED((NP, DW), jnp.float32),
            pltpu.VMEM((CH,), jnp.int32),
            pltpu.VMEM((CH, DW), jnp.float32),
            pltpu.VMEM((STRIPE, DW), jnp.float32),
        ],
    )
    def k(dsti_hbm, dstj_hbm, out_hbm, acc_i, acc_j, idxv, ones, zbuf):
        cid = lax.axis_index("c")
        sid = lax.axis_index("s")
        wid = cid * 16 + sid

        def fill_ones(r, carry):
            ones[r, pl.ds(0, DW)] = jnp.ones((16,), jnp.float32)
            return carry

        lax.fori_loop(0, CH, fill_ones, 0)

        def fill_zero(r, carry):
            zbuf[r, pl.ds(0, DW)] = jnp.zeros((16,), jnp.float32)
            return carry

        lax.fori_loop(0, STRIPE, fill_zero, 0)
        pltpu.sync_copy(zbuf, acc_i.at[pl.ds(sid * STRIPE, STRIPE)])
        pltpu.sync_copy(zbuf, acc_j.at[pl.ds(sid * STRIPE, STRIPE)])
        plsc.subcore_barrier()

        ebase = wid * (KCH * CH)
        for dref, aref in ((dsti_hbm, acc_i), (dstj_hbm, acc_j)):
            def body(kk, carry, dref=dref, aref=aref):
                pltpu.sync_copy(dref.at[pl.ds(ebase + kk * CH, CH)], idxv)
                pltpu.sync_copy(ones, aref.at[idxv], add=True)
                return carry

            lax.fori_loop(0, KCH, body, 0)
        plsc.subcore_barrier()

        pltpu.sync_copy(
            acc_i.at[pl.ds(sid * STRIPE, STRIPE)],
            out_hbm.at[pl.ds((0 + cid) * NP + sid * STRIPE, STRIPE)])
        pltpu.sync_copy(
            acc_j.at[pl.ds(sid * STRIPE, STRIPE)],
            out_hbm.at[pl.ds((2 + cid) * NP + sid * STRIPE, STRIPE)])

    return k


def _make_segsum_kernel(F):
    mesh = plsc.VectorSubcoreMesh(core_axis_name="c", subcore_axis_name="s")

    @functools.partial(
        pl.kernel,
        out_type=jax.ShapeDtypeStruct((2 * NP, F), jnp.float32),
        mesh=mesh,
        compiler_params=pltpu.CompilerParams(use_tc_tiling_on_sc=False),
        scratch_types=[
            pltpu.VMEM_SHARED((NP, F), jnp.float32),
            pltpu.VMEM((CH,), jnp.int32),
            pltpu.VMEM((CH,), jnp.int32),
            pltpu.VMEM((CH, F), jnp.float32),
            pltpu.VMEM((STRIPE, F), jnp.float32),
            pltpu.SemaphoreType.DMA,
        ],
    )
    def k(y_hbm, src_hbm, dst_hbm, out_hbm, acc, srcv, dstv, rows, zbuf, sem):
        cid = lax.axis_index("c")
        sid = lax.axis_index("s")
        wid = cid * 16 + sid

        def fill_zero(r, carry):
            for c in range(F // 16):
                zbuf[r, pl.ds(c * 16, 16)] = jnp.zeros((16,), jnp.float32)
            return carry

        lax.fori_loop(0, STRIPE, fill_zero, 0)
        pltpu.sync_copy(zbuf, acc.at[pl.ds(sid * STRIPE, STRIPE)])
        plsc.subcore_barrier()

        ebase = wid * (KCH * CH)

        def body(kk, carry):
            off = ebase + kk * CH
            pltpu.sync_copy(src_hbm.at[pl.ds(off, CH)], srcv)
            pltpu.sync_copy(dst_hbm.at[pl.ds(off, CH)], dstv)
            pltpu.async_copy(y_hbm.at[srcv], rows, sem).wait()
            pltpu.sync_copy(rows, acc.at[dstv], add=True)
            return carry

        lax.fori_loop(0, KCH, body, 0)
        plsc.subcore_barrier()

        pltpu.sync_copy(
            acc.at[pl.ds(sid * STRIPE, STRIPE)],
            out_hbm.at[pl.ds(cid * NP + sid * STRIPE, STRIPE)])

    return k


def _dinv_call(parts):
    def body(p_ref, o_ref):
        di = lax.rsqrt(1.0 + p_ref[0:NP, 0:1] + p_ref[NP:2 * NP, 0:1])
        dj = lax.rsqrt(1.0 + p_ref[2 * NP:3 * NP, 0:1] + p_ref[3 * NP:4 * NP, 0:1])
        o_ref[0:NP] = di
        o_ref[NP:2 * NP] = dj

    return pl.pallas_call(
        body, out_shape=jax.ShapeDtypeStruct((2 * NP, 1), jnp.float32))(parts)


def _pre_call(x, W, dinv):
    F = W.shape[1]

    def body(x_ref, w_ref, d_ref, o_ref):
        y = d_ref[0:N] * jnp.dot(x_ref[...], w_ref[...],
                                 preferred_element_type=jnp.float32)
        o_ref[0:N] = y
        o_ref[N:NP] = jnp.zeros((NP - N, F), jnp.float32)

    return pl.pallas_call(
        body, out_shape=jax.ShapeDtypeStruct((NP, F), jnp.float32))(x, W, dinv)


def _make_mid_call(F, Fn):
    def body(a_ref, y_ref, d_ref, b_ref, w_ref, o_ref):
        s = a_ref[0:NP] + a_ref[NP:2 * NP] + y_ref[...]
        z = jnp.maximum(d_ref[...] * s + b_ref[...], 0.0)
        o_ref[0:N] = d_ref[0:N] * jnp.dot(
            z[0:N], w_ref[...], preferred_element_type=jnp.float32)
        o_ref[N:NP] = jnp.zeros((NP - N, Fn), jnp.float32)

    def call(acc, y, dinv, b, W):
        return pl.pallas_call(
            body, out_shape=jax.ShapeDtypeStruct((NP, Fn), jnp.float32))(
                acc, y, dinv, b, W)

    return call


def _att_call(acc, y, dinv, b, W_att):
    def body(a_ref, y_ref, d_ref, b_ref, w_ref, o_ref):
        enc = d_ref[...] * (a_ref[0:NP] + a_ref[NP:2 * NP] + y_ref[...]) + b_ref[...]
        encn = enc[0:N]
        m = jnp.sum(encn, axis=0, keepdims=True) * (1.0 / N)
        c = jnp.tanh(jnp.dot(m, w_ref[...], preferred_element_type=jnp.float32))
        s = jax.nn.sigmoid(jnp.sum(encn * c, axis=1, keepdims=True))
        o_ref[...] = jnp.sum(encn * s, axis=0, keepdims=True)

    return pl.pallas_call(
        body, out_shape=jax.ShapeDtypeStruct((1, F3), jnp.float32))(
            acc, y, dinv, b, W_att)


def _head_call(hi, hj, W_ntn, V_ntn_t, b_ntn, W_m0, b_m0, W_m1, b_m1,
               W_m2, b_m2, W_m3, b_m3, W_s, b_s):
    def body(hi_ref, hj_ref, wn_ref, vt_ref, bn_ref, w0_ref, c0_ref, w1_ref,
             c1_ref, w2_ref, c2_ref, w3_ref, c3_ref, ws_ref, cs_ref, o_ref):
        hiv = hi_ref[...]
        hjv = hj_ref[...]
        ks = lax.broadcasted_iota(jnp.int32, (1, NTN_SLICES), 1)
        bil = jnp.zeros((1, NTN_SLICES), jnp.float32)
        for k in range(NTN_SLICES):
            wk = wn_ref[k]
            val = jnp.sum(jnp.dot(hiv, wk, preferred_element_type=jnp.float32)
                          * hjv)
            bil = bil + jnp.where(ks == k, val, 0.0)
        cat = jnp.concatenate([hiv, hjv], axis=1)
        lin = jnp.dot(cat, vt_ref[...], preferred_element_type=jnp.float32)
        inter = jnp.tanh(bil + lin + bn_ref[...])
        sim = jax.nn.sigmoid(jnp.sum(hiv * hjv))
        binx = jnp.clip((sim * HIST_BINS).astype(jnp.int32), 0, HIST_BINS - 1)
        hist = jnp.where(ks == binx, 1.0, 0.0)
        feat = jnp.concatenate([inter, hist], axis=1)
        for w_ref, c_ref in ((w0_ref, c0_ref), (w1_ref, c1_ref),
                             (w2_ref, c2_ref), (w3_ref, c3_ref)):
            feat = jnp.maximum(
                jnp.dot(feat, w_ref[...], preferred_element_type=jnp.float32)
                + c_ref[...], 0.0)
        o_ref[...] = jnp.dot(feat, ws_ref[...],
                             preferred_element_type=jnp.float32) + cs_ref[...]

    return pl.pallas_call(
        body, out_shape=jax.ShapeDtypeStruct((1, 1), jnp.float32))(
            hi, hj, W_ntn, V_ntn_t, b_ntn, W_m0, b_m0, W_m1, b_m1,
            W_m2, b_m2, W_m3, b_m3, W_s, b_s)


_deg_kernel = _make_degree_kernel()
_seg64 = _make_segsum_kernel(F1)
_seg32 = _make_segsum_kernel(F2)
_seg16 = _make_segsum_kernel(F3)
_mid_64_32 = _make_mid_call(F1, F2)
_mid_32_16 = _make_mid_call(F2, F3)


def kernel(x_i, x_j, edge_index_i, edge_index_j, W1, b1, W2, b2, W3, b3,
           W_att, W_ntn, V_ntn, b_ntn, W_m0, b_m0, W_m1, b_m1, W_m2, b_m2,
           W_m3, b_m3, W_s, b_s):
    pad_idx = jnp.full((EP - E,), N, jnp.int32)
    src_i = jnp.concatenate([edge_index_i[0], pad_idx])
    dst_i = jnp.concatenate([edge_index_i[1], pad_idx])
    src_j = jnp.concatenate([edge_index_j[0], pad_idx])
    dst_j = jnp.concatenate([edge_index_j[1], pad_idx])

    deg_parts = _deg_kernel(dst_i, dst_j)
    dinv = _dinv_call(deg_parts)
    dinv_i = lax.slice(dinv, (0, 0), (NP, 1))
    dinv_j = lax.slice(dinv, (NP, 0), (2 * NP, 1))

    b1r = b1.reshape(1, F1)
    b2r = b2.reshape(1, F2)
    b3r = b3.reshape(1, F3)

    def encode(x, srcp, dstp, dinv_g):
        y1 = _pre_call(x, W1, dinv_g)
        a1 = _seg64(y1, srcp, dstp)
        y2 = _mid_64_32(a1, y1, dinv_g, b1r, W2)
        a2 = _seg32(y2, srcp, dstp)
        y3 = _mid_32_16(a2, y2, dinv_g, b2r, W3)
        a3 = _seg16(y3, srcp, dstp)
        return _att_call(a3, y3, dinv_g, b3r, W_att)

    hi = encode(x_i, src_i, dst_i, dinv_i)
    hj = encode(x_j, src_j, dst_j, dinv_j)

    out = _head_call(hi, hj, W_ntn, V_ntn.T, b_ntn.reshape(1, NTN_SLICES),
                     W_m0, b_m0.reshape(1, 32), W_m1, b_m1.reshape(1, 16),
                     W_m2, b_m2.reshape(1, 8), W_m3, b_m3.reshape(1, 4),
                     W_s, b_s.reshape(1, 1))
    return out.reshape(1)


# speedup vs baseline: 18.2493x; 1.4197x over previous
import functools

import jax
import jax.numpy as jnp
from jax import lax
from jax.experimental import pallas as pl
from jax.experimental.pallas import tpu as pltpu
from jax.experimental.pallas import tpu_sc as plsc

N = 10000
NP = 10112
E = 320000
D_IN = 128
F1, F2, F3 = 64, 32, 16
NTN_SLICES = 16
HIST_BINS = 16

NWORK = 32
CH = 128
KCH = 80
EP = NWORK * KCH * CH
ER = EP // CH
STRIPE = NP // 16


def _make_degree_kernel():
    DW = 8
    mesh = plsc.VectorSubcoreMesh(core_axis_name="c", subcore_axis_name="s")

    @functools.partial(
        pl.kernel,
        out_type=jax.ShapeDtypeStruct((4 * NP, DW), jnp.float32),
        mesh=mesh,
        compiler_params=pltpu.CompilerParams(use_tc_tiling_on_sc=False),
        scratch_types=[
            pltpu.VMEM_SHARED((NP, DW), jnp.float32),
            pltpu.VMEM_SHARED((NP, DW), jnp.float32),
            pltpu.VMEM((KCH, CH), jnp.int32),
            pltpu.VMEM((KCH, CH), jnp.int32),
            pltpu.VMEM((CH, DW), jnp.float32),
            pltpu.VMEM((STRIPE, DW), jnp.float32),
            pltpu.SemaphoreType.DMA,
        ],
    )
    def k(dsti_hbm, dstj_hbm, out_hbm, acc_i, acc_j, dbi, dbj, ones, zbuf,
          sem):
        cid = lax.axis_index("c")
        sid = lax.axis_index("s")
        wid = cid * 16 + sid

        pltpu.sync_copy(dsti_hbm.at[pl.ds(wid * KCH, KCH)], dbi)
        pltpu.sync_copy(dstj_hbm.at[pl.ds(wid * KCH, KCH)], dbj)

        def fill_ones(r, carry):
            ones[r, pl.ds(0, DW)] = jnp.ones((DW,), jnp.float32)
            return carry

        def fill_zero(r, carry):
            zbuf[r, pl.ds(0, DW)] = jnp.zeros((DW,), jnp.float32)
            return carry

        lax.fori_loop(0, CH, fill_ones, 0)
        lax.fori_loop(0, STRIPE, fill_zero, 0)
        pltpu.sync_copy(zbuf, acc_i.at[pl.ds(sid * STRIPE, STRIPE)])
        pltpu.sync_copy(zbuf, acc_j.at[pl.ds(sid * STRIPE, STRIPE)])
        plsc.subcore_barrier()

        for db, aref in ((dbi, acc_i), (dbj, acc_j)):
            def body(g, carry, db=db, aref=aref):
                for u in range(4):
                    pltpu.async_copy(ones, aref.at[db.at[g * 4 + u]], sem,
                                     add=True)
                for u in range(4):
                    pltpu.make_async_copy(ones, aref.at[db.at[g * 4 + u]],
                                          sem).wait()
                return carry

            lax.fori_loop(0, KCH // 4, body, 0)
        plsc.subcore_barrier()

        pltpu.sync_copy(
            acc_i.at[pl.ds(sid * STRIPE, STRIPE)],
            out_hbm.at[pl.ds((0 + cid) * NP + sid * STRIPE, STRIPE)])
        pltpu.sync_copy(
            acc_j.at[pl.ds(sid * STRIPE, STRIPE)],
            out_hbm.at[pl.ds((2 + cid) * NP + sid * STRIPE, STRIPE)])

    return k


def _make_segsum_kernel(F):
    mesh = plsc.VectorSubcoreMesh(core_axis_name="c", subcore_axis_name="s")

    @functools.partial(
        pl.kernel,
        out_type=jax.ShapeDtypeStruct((2 * NP, F), jnp.float32),
        mesh=mesh,
        compiler_params=pltpu.CompilerParams(use_tc_tiling_on_sc=False),
        scratch_types=[
            pltpu.VMEM_SHARED((NP, F), jnp.float32),
            pltpu.VMEM((KCH, CH), jnp.int32),
            pltpu.VMEM((KCH, CH), jnp.int32),
            pltpu.VMEM((CH, F), jnp.float32),
            pltpu.VMEM((CH, F), jnp.float32),
            pltpu.VMEM((STRIPE, F), jnp.float32),
            pltpu.SemaphoreType.DMA,
            pltpu.SemaphoreType.DMA,
        ],
    )
    def k(y_hbm, src_hbm, dst_hbm, out_hbm, acc, srcb, dstb, rows0, rows1,
          zbuf, sem0, sem1):
        cid = lax.axis_index("c")
        sid = lax.axis_index("s")
        wid = cid * 16 + sid

        pltpu.sync_copy(src_hbm.at[pl.ds(wid * KCH, KCH)], srcb)
        pltpu.sync_copy(dst_hbm.at[pl.ds(wid * KCH, KCH)], dstb)

        def fill_zero(r, carry):
            for c in range(F // 16):
                zbuf[r, pl.ds(c * 16, 16)] = jnp.zeros((16,), jnp.float32)
            return carry

        lax.fori_loop(0, STRIPE, fill_zero, 0)
        pltpu.sync_copy(zbuf, acc.at[pl.ds(sid * STRIPE, STRIPE)])
        plsc.subcore_barrier()

        pltpu.async_copy(y_hbm.at[srcb.at[0]], rows0, sem0)

        def body(u, carry):
            k0 = 2 * u
            pltpu.async_copy(y_hbm.at[srcb.at[k0 + 1]], rows1, sem1)
            pltpu.make_async_copy(y_hbm.at[srcb.at[k0]], rows0, sem0).wait()
            pltpu.sync_copy(rows0, acc.at[dstb.at[k0]], add=True)

            @pl.when(u + 1 < KCH // 2)
            def _():
                pltpu.async_copy(y_hbm.at[srcb.at[k0 + 2]], rows0, sem0)

            pltpu.make_async_copy(y_hbm.at[srcb.at[k0 + 1]], rows1,
                                  sem1).wait()
            pltpu.sync_copy(rows1, acc.at[dstb.at[k0 + 1]], add=True)
            return carry

        lax.fori_loop(0, KCH // 2, body, 0)
        plsc.subcore_barrier()

        pltpu.sync_copy(
            acc.at[pl.ds(sid * STRIPE, STRIPE)],
            out_hbm.at[pl.ds(cid * NP + sid * STRIPE, STRIPE)])

    return k


def _dinv_call(parts):
    def body(p_ref, o_ref):
        di = lax.rsqrt(1.0 + p_ref[0:NP, 0:1] + p_ref[NP:2 * NP, 0:1])
        dj = lax.rsqrt(1.0 + p_ref[2 * NP:3 * NP, 0:1] + p_ref[3 * NP:4 * NP, 0:1])
        o_ref[0:NP] = di
        o_ref[NP:2 * NP] = dj

    return pl.pallas_call(
        body, out_shape=jax.ShapeDtypeStruct((2 * NP, 1), jnp.float32))(parts)


def _xw_call(x, W):
    def body(x_ref, w_ref, o_ref):
        o_ref[...] = jnp.dot(x_ref[...], w_ref[...],
                             preferred_element_type=jnp.float32)

    return pl.pallas_call(
        body, out_shape=jax.ShapeDtypeStruct((2 * N, F1), jnp.float32))(x, W)


def _scale_call(xw, dinv):
    def body(x_ref, d_ref, o_ref):
        o_ref[0:N] = d_ref[0:N] * x_ref[...]
        o_ref[N:NP] = jnp.zeros((NP - N, F1), jnp.float32)

    return pl.pallas_call(
        body, out_shape=jax.ShapeDtypeStruct((NP, F1), jnp.float32))(xw, dinv)


def _make_mid_call(F, Fn):
    def body(a_ref, y_ref, d_ref, b_ref, w_ref, o_ref):
        s = a_ref[0:NP] + a_ref[NP:2 * NP] + y_ref[...]
        z = jnp.maximum(d_ref[...] * s + b_ref[...], 0.0)
        o_ref[0:N] = d_ref[0:N] * jnp.dot(
            z[0:N], w_ref[...], preferred_element_type=jnp.float32)
        o_ref[N:NP] = jnp.zeros((NP - N, Fn), jnp.float32)

    def call(acc, y, dinv, b, W):
        return pl.pallas_call(
            body, out_shape=jax.ShapeDtypeStruct((NP, Fn), jnp.float32))(
                acc, y, dinv, b, W)

    return call


def _att_call(acc, y, dinv, b, W_att):
    def body(a_ref, y_ref, d_ref, b_ref, w_ref, o_ref):
        enc = d_ref[...] * (a_ref[0:NP] + a_ref[NP:2 * NP] + y_ref[...]) + b_ref[...]
        encn = enc[0:N]
        m = jnp.sum(encn, axis=0, keepdims=True) * (1.0 / N)
        c = jnp.tanh(jnp.dot(m, w_ref[...], preferred_element_type=jnp.float32))
        s = jax.nn.sigmoid(jnp.sum(encn * c, axis=1, keepdims=True))
        o_ref[...] = jnp.sum(encn * s, axis=0, keepdims=True)

    return pl.pallas_call(
        body, out_shape=jax.ShapeDtypeStruct((1, F3), jnp.float32))(
            acc, y, dinv, b, W_att)


def _head_call(hi, hj, W_ntn, V_ntn_t, b_ntn, W_m0, b_m0, W_m1, b_m1,
               W_m2, b_m2, W_m3, b_m3, W_s, b_s):
    def body(hi_ref, hj_ref, wn_ref, vt_ref, bn_ref, w0_ref, c0_ref, w1_ref,
             c1_ref, w2_ref, c2_ref, w3_ref, c3_ref, ws_ref, cs_ref, o_ref):
        hiv = hi_ref[...]
        hjv = hj_ref[...]
        ks = lax.broadcasted_iota(jnp.int32, (1, NTN_SLICES), 1)
        bil = jnp.zeros((1, NTN_SLICES), jnp.float32)
        for k in range(NTN_SLICES):
            wk = wn_ref[k]
            val = jnp.sum(jnp.dot(hiv, wk, preferred_element_type=jnp.float32)
                          * hjv)
            bil = bil + jnp.where(ks == k, val, 0.0)
        cat = jnp.concatenate([hiv, hjv], axis=1)
        lin = jnp.dot(cat, vt_ref[...], preferred_element_type=jnp.float32)
        inter = jnp.tanh(bil + lin + bn_ref[...])
        sim = jax.nn.sigmoid(jnp.sum(hiv * hjv))
        binx = jnp.clip((sim * HIST_BINS).astype(jnp.int32), 0, HIST_BINS - 1)
        hist = jnp.where(ks == binx, 1.0, 0.0)
        feat = jnp.concatenate([inter, hist], axis=1)
        for w_ref, c_ref in ((w0_ref, c0_ref), (w1_ref, c1_ref),
                             (w2_ref, c2_ref), (w3_ref, c3_ref)):
            feat = jnp.maximum(
                jnp.dot(feat, w_ref[...], preferred_element_type=jnp.float32)
                + c_ref[...], 0.0)
        o_ref[...] = jnp.dot(feat, ws_ref[...],
                             preferred_element_type=jnp.float32) + cs_ref[...]

    return pl.pallas_call(
        body, out_shape=jax.ShapeDtypeStruct((1, 1), jnp.float32))(
            hi, hj, W_ntn, V_ntn_t, b_ntn, W_m0, b_m0, W_m1, b_m1,
            W_m2, b_m2, W_m3, b_m3, W_s, b_s)


_deg_kernel = _make_degree_kernel()
_seg64 = _make_segsum_kernel(F1)
_seg32 = _make_segsum_kernel(F2)
_seg16 = _make_segsum_kernel(F3)
_mid_64_32 = _make_mid_call(F1, F2)
_mid_32_16 = _make_mid_call(F2, F3)


def _pad_edges(e):
    return jnp.concatenate([e, jnp.full((EP - E,), N, jnp.int32)]).reshape(
        ER // NWORK * NWORK, CH)


def kernel(x_i, x_j, edge_index_i, edge_index_j, W1, b1, W2, b2, W3, b3,
           W_att, W_ntn, V_ntn, b_ntn, W_m0, b_m0, W_m1, b_m1, W_m2, b_m2,
           W_m3, b_m3, W_s, b_s):
    src_i = _pad_edges(edge_index_i[0])
    dst_i = _pad_edges(edge_index_i[1])
    src_j = _pad_edges(edge_index_j[0])
    dst_j = _pad_edges(edge_index_j[1])

    deg_parts = _deg_kernel(dst_i, dst_j)
    xw1 = _xw_call(jnp.concatenate([x_i, x_j], axis=0), W1)
    dinv = _dinv_call(deg_parts)
    dinv_i = lax.slice(dinv, (0, 0), (NP, 1))
    dinv_j = lax.slice(dinv, (NP, 0), (2 * NP, 1))

    b1r = b1.reshape(1, F1)
    b2r = b2.reshape(1, F2)
    b3r = b3.reshape(1, F3)

    def encode(xw_g, srcp, dstp, dinv_g):
        y1 = _scale_call(xw_g, dinv_g)
        a1 = _seg64(y1, srcp, dstp)
        y2 = _mid_64_32(a1, y1, dinv_g, b1r, W2)
        a2 = _seg32(y2, srcp, dstp)
        y3 = _mid_32_16(a2, y2, dinv_g, b2r, W3)
        a3 = _seg16(y3, srcp, dstp)
        return _att_call(a3, y3, dinv_g, b3r, W_att)

    hi = encode(lax.slice(xw1, (0, 0), (N, F1)), src_i, dst_i, dinv_i)
    hj = encode(lax.slice(xw1, (N, 0), (2 * N, F1)), src_j, dst_j, dinv_j)

    out = _head_call(hi, hj, W_ntn, V_ntn.T, b_ntn.reshape(1, NTN_SLICES),
                     W_m0, b_m0.reshape(1, 32), W_m1, b_m1.reshape(1, 16),
                     W_m2, b_m2.reshape(1, 8), W_m3, b_m3.reshape(1, 4),
                     W_s, b_s.reshape(1, 1))
    return out.reshape(1)


# speedup vs baseline: 34.1162x; 1.8694x over previous
import functools

import jax
import jax.numpy as jnp
from jax import lax
from jax.experimental import pallas as pl
from jax.experimental.pallas import tpu as pltpu
from jax.experimental.pallas import tpu_sc as plsc

N = 10000
NP = 10112
NT = 2 * NP
E = 320000
D_IN = 128
F1, F2, F3 = 64, 32, 16
NTN_SLICES = 16
HIST_BINS = 16

NWORK = 32
CH = 128
KCH = 80
EP = NWORK * KCH * CH
GR = EP // CH
ER = 2 * GR
STRIPE = NP // 16


def _make_degree_kernel():
    DW = 8
    mesh = plsc.VectorSubcoreMesh(core_axis_name="c", subcore_axis_name="s")

    @functools.partial(
        pl.kernel,
        out_type=jax.ShapeDtypeStruct((4 * NP, DW), jnp.float32),
        mesh=mesh,
        compiler_params=pltpu.CompilerParams(use_tc_tiling_on_sc=False),
        scratch_types=[
            pltpu.VMEM_SHARED((NP, DW), jnp.float32),
            pltpu.VMEM((KCH, CH), jnp.int32),
            pltpu.VMEM((CH, DW), jnp.float32),
            pltpu.VMEM((STRIPE, DW), jnp.float32),
            pltpu.SemaphoreType.DMA,
        ],
    )
    def k(dst_hbm, out_hbm, acc, db, ones, zbuf, sem):
        cid = lax.axis_index("c")
        sid = lax.axis_index("s")
        wid = cid * 16 + sid

        def fill_ones(r, carry):
            ones[r, pl.ds(0, DW)] = jnp.ones((DW,), jnp.float32)
            return carry

        def fill_zero(r, carry):
            zbuf[r, pl.ds(0, DW)] = jnp.zeros((DW,), jnp.float32)
            return carry

        lax.fori_loop(0, CH, fill_ones, 0)
        lax.fori_loop(0, STRIPE, fill_zero, 0)

        for g in range(2):
            pltpu.sync_copy(dst_hbm.at[pl.ds(g * GR + wid * KCH, KCH)], db)
            pltpu.sync_copy(zbuf, acc.at[pl.ds(sid * STRIPE, STRIPE)])
            plsc.subcore_barrier()

            def body(t, carry):
                for u in range(4):
                    pltpu.async_copy(ones, acc.at[db.at[t * 4 + u]],
                                     sem, add=True)
                for u in range(4):
                    pltpu.make_async_copy(
                        ones, acc.at[db.at[t * 4 + u]], sem).wait()
                return carry

            lax.fori_loop(0, KCH // 4, body, 0)
            plsc.subcore_barrier()

            pltpu.sync_copy(
                acc.at[pl.ds(sid * STRIPE, STRIPE)],
                out_hbm.at[pl.ds((g * 2 + cid) * NP + sid * STRIPE, STRIPE)])

    return k


def _make_segsum_kernel(F):
    mesh = plsc.VectorSubcoreMesh(core_axis_name="c", subcore_axis_name="s")

    @functools.partial(
        pl.kernel,
        out_type=jax.ShapeDtypeStruct((4 * NP, F), jnp.float32),
        mesh=mesh,
        compiler_params=pltpu.CompilerParams(use_tc_tiling_on_sc=False),
        scratch_types=[
            pltpu.VMEM_SHARED((NP, F), jnp.float32),
            pltpu.VMEM((KCH, CH), jnp.int32),
            pltpu.VMEM((KCH, CH), jnp.int32),
            pltpu.VMEM((CH, F), jnp.float32),
            pltpu.VMEM((CH, F), jnp.float32),
            pltpu.VMEM((STRIPE, F), jnp.float32),
            pltpu.SemaphoreType.DMA,
            pltpu.SemaphoreType.DMA,
        ],
    )
    def k(y_hbm, src_hbm, dst_hbm, out_hbm, acc, srcb, dstb, rows0, rows1,
          zbuf, sem0, sem1):
        cid = lax.axis_index("c")
        sid = lax.axis_index("s")
        wid = cid * 16 + sid

        def fill_zero(r, carry):
            for c in range(F // 16):
                zbuf[r, pl.ds(c * 16, 16)] = jnp.zeros((16,), jnp.float32)
            return carry

        lax.fori_loop(0, STRIPE, fill_zero, 0)

        for g in range(2):
            pltpu.sync_copy(src_hbm.at[pl.ds(g * GR + wid * KCH, KCH)], srcb)
            pltpu.sync_copy(dst_hbm.at[pl.ds(g * GR + wid * KCH, KCH)], dstb)
            pltpu.sync_copy(zbuf, acc.at[pl.ds(sid * STRIPE, STRIPE)])
            plsc.subcore_barrier()

            pltpu.async_copy(y_hbm.at[srcb.at[0]], rows0, sem0)

            def body(u, carry):
                k0 = 2 * u
                pltpu.async_copy(y_hbm.at[srcb.at[k0 + 1]], rows1, sem1)
                pltpu.make_async_copy(y_hbm.at[srcb.at[k0]], rows0,
                                      sem0).wait()
                pltpu.sync_copy(rows0, acc.at[dstb.at[k0]], add=True)

                @pl.when(2 * u + 2 < KCH)
                def _():
                    pltpu.async_copy(y_hbm.at[srcb.at[k0 + 2]], rows0, sem0)

                pltpu.make_async_copy(y_hbm.at[srcb.at[k0 + 1]], rows1,
                                      sem1).wait()
                pltpu.sync_copy(rows1, acc.at[dstb.at[k0 + 1]], add=True)
                return carry

            lax.fori_loop(0, KCH // 2, body, 0)
            plsc.subcore_barrier()

            pltpu.sync_copy(
                acc.at[pl.ds(sid * STRIPE, STRIPE)],
                out_hbm.at[pl.ds((g * 2 + cid) * NP + sid * STRIPE, STRIPE)])

    return k


def _xw_call(x, W):
    def body(x_ref, w_ref, o_ref):
        o_ref[...] = jnp.dot(x_ref[...], w_ref[...],
                             preferred_element_type=jnp.float32)

    return pl.pallas_call(
        body, out_shape=jax.ShapeDtypeStruct((2 * N, F1), jnp.float32))(x, W)


def _dinv_scale_call(parts, xw):
    def body(p_ref, x_ref, d_ref, y_ref):
        di = lax.rsqrt(1.0 + p_ref[0:NP, 0:1] + p_ref[NP:2 * NP, 0:1])
        dj = lax.rsqrt(1.0 + p_ref[2 * NP:3 * NP, 0:1]
                       + p_ref[3 * NP:4 * NP, 0:1])
        d_ref[0:NP] = di
        d_ref[NP:NT] = dj
        y_ref[0:N] = di[0:N] * x_ref[0:N]
        y_ref[N:NP] = jnp.zeros((NP - N, F1), jnp.float32)
        y_ref[NP:NP + N] = dj[0:N] * x_ref[N:2 * N]
        y_ref[NP + N:NT] = jnp.zeros((NP - N, F1), jnp.float32)

    return pl.pallas_call(
        body,
        out_shape=(jax.ShapeDtypeStruct((NT, 1), jnp.float32),
                   jax.ShapeDtypeStruct((NT, F1), jnp.float32)))(parts, xw)


def _make_mid_call(F, Fn):
    def body(a_ref, y_ref, d_ref, b_ref, w_ref, o_ref):
        s_i = a_ref[0:NP] + a_ref[NP:2 * NP] + y_ref[0:NP]
        s_j = a_ref[2 * NP:3 * NP] + a_ref[3 * NP:4 * NP] + y_ref[NP:NT]
        s = jnp.concatenate([s_i, s_j], axis=0)
        z = jnp.maximum(d_ref[...] * s + b_ref[...], 0.0)
        o_ref[...] = d_ref[...] * jnp.dot(
            z, w_ref[...], preferred_element_type=jnp.float32)
        o_ref[N:NP] = jnp.zeros((NP - N, Fn), jnp.float32)
        o_ref[NP + N:NT] = jnp.zeros((NP - N, Fn), jnp.float32)

    def call(acc, y, dinv, b, W):
        return pl.pallas_call(
            body, out_shape=jax.ShapeDtypeStruct((NT, Fn), jnp.float32))(
                acc, y, dinv, b, W)

    return call


def _att_call(acc, y, dinv, b, W_att):
    def body(a_ref, y_ref, d_ref, b_ref, w_ref, o_ref):
        s_i = a_ref[0:NP] + a_ref[NP:2 * NP] + y_ref[0:NP]
        s_j = a_ref[2 * NP:3 * NP] + a_ref[3 * NP:4 * NP] + y_ref[NP:NT]
        enc = d_ref[...] * jnp.concatenate([s_i, s_j], axis=0) + b_ref[...]
        for g in range(2):
            encn = enc[g * NP:g * NP + N]
            m = jnp.sum(encn, axis=0, keepdims=True) * (1.0 / N)
            c = jnp.tanh(jnp.dot(m, w_ref[...],
                                 preferred_element_type=jnp.float32))
            s = jax.nn.sigmoid(jnp.sum(encn * c, axis=1, keepdims=True))
            o_ref[g:g + 1, :] = jnp.sum(encn * s, axis=0, keepdims=True)

    return pl.pallas_call(
        body, out_shape=jax.ShapeDtypeStruct((2, F3), jnp.float32))(
            acc, y, dinv, b, W_att)


def _head_call(h, W_ntn, V_ntn_t, b_ntn, W_m0, b_m0, W_m1, b_m1,
               W_m2, b_m2, W_m3, b_m3, W_s, b_s):
    def body(h_ref, wn_ref, vt_ref, bn_ref, w0_ref, c0_ref, w1_ref,
             c1_ref, w2_ref, c2_ref, w3_ref, c3_ref, ws_ref, cs_ref, o_ref):
        hiv = h_ref[0:1]
        hjv = h_ref[1:2]
        ks = lax.broadcasted_iota(jnp.int32, (1, NTN_SLICES), 1)
        bil = jnp.zeros((1, NTN_SLICES), jnp.float32)
        for k in range(NTN_SLICES):
            wk = wn_ref[k]
            val = jnp.sum(jnp.dot(hiv, wk, preferred_element_type=jnp.float32)
                          * hjv)
            bil = bil + jnp.where(ks == k, val, 0.0)
        cat = jnp.concatenate([hiv, hjv], axis=1)
        lin = jnp.dot(cat, vt_ref[...], preferred_element_type=jnp.float32)
        inter = jnp.tanh(bil + lin + bn_ref[...])
        sim = jax.nn.sigmoid(jnp.sum(hiv * hjv))
        binx = jnp.clip((sim * HIST_BINS).astype(jnp.int32), 0, HIST_BINS - 1)
        hist = jnp.where(ks == binx, 1.0, 0.0)
        feat = jnp.concatenate([inter, hist], axis=1)
        for w_ref, c_ref in ((w0_ref, c0_ref), (w1_ref, c1_ref),
                             (w2_ref, c2_ref), (w3_ref, c3_ref)):
            feat = jnp.maximum(
                jnp.dot(feat, w_ref[...], preferred_element_type=jnp.float32)
                + c_ref[...], 0.0)
        o_ref[...] = jnp.dot(feat, ws_ref[...],
                             preferred_element_type=jnp.float32) + cs_ref[...]

    return pl.pallas_call(
        body, out_shape=jax.ShapeDtypeStruct((1, 1), jnp.float32))(
            h, W_ntn, V_ntn_t, b_ntn, W_m0, b_m0, W_m1, b_m1,
            W_m2, b_m2, W_m3, b_m3, W_s, b_s)


_deg_kernel = _make_degree_kernel()
_seg64 = _make_segsum_kernel(F1)
_seg32 = _make_segsum_kernel(F2)
_seg16 = _make_segsum_kernel(F3)
_mid_64_32 = _make_mid_call(F1, F2)
_mid_32_16 = _make_mid_call(F2, F3)


def _pad_edges(e, base):
    fill = N + jnp.arange(EP - E, dtype=jnp.int32) % (NP - N)
    return jnp.concatenate([e + base, fill + base])


def kernel(x_i, x_j, edge_index_i, edge_index_j, W1, b1, W2, b2, W3, b3,
           W_att, W_ntn, V_ntn, b_ntn, W_m0, b_m0, W_m1, b_m1, W_m2, b_m2,
           W_m3, b_m3, W_s, b_s):
    src = jnp.concatenate([_pad_edges(edge_index_i[0], 0),
                           _pad_edges(edge_index_j[0], NP)]).reshape(ER, CH)
    dst = jnp.concatenate([_pad_edges(edge_index_i[1], 0),
                           _pad_edges(edge_index_j[1], 0)]).reshape(ER, CH)

    deg_parts = _deg_kernel(dst)
    xw1 = _xw_call(jnp.concatenate([x_i, x_j], axis=0), W1)
    dinv, y1 = _dinv_scale_call(deg_parts, xw1)

    a1 = _seg64(y1, src, dst)
    y2 = _mid_64_32(a1, y1, dinv, b1.reshape(1, F1), W2)
    a2 = _seg32(y2, src, dst)
    y3 = _mid_32_16(a2, y2, dinv, b2.reshape(1, F2), W3)
    a3 = _seg16(y3, src, dst)
    h = _att_call(a3, y3, dinv, b3.reshape(1, F3), W_att)

    out = _head_call(h, W_ntn, V_ntn.T, b_ntn.reshape(1, NTN_SLICES),
                     W_m0, b_m0.reshape(1, 32), W_m1, b_m1.reshape(1, 16),
                     W_m2, b_m2.reshape(1, 8), W_m3, b_m3.reshape(1, 4),
                     W_s, b_s.reshape(1, 1))
    return out.reshape(1)


# speedup vs baseline: 39.5853x; 1.1603x over previous
import functools

import jax
import jax.numpy as jnp
from jax import lax
from jax.experimental import pallas as pl
from jax.experimental.pallas import tpu as pltpu
from jax.experimental.pallas import tpu_sc as plsc

N = 10000
NP = 10112
NT = 2 * NP
E = 320000
D_IN = 128
F1, F2, F3 = 64, 32, 16
NTN_SLICES = 16
HIST_BINS = 16

NWORK = 32
CH = 128
KCH = 80
EP = NWORK * KCH * CH
GR = EP // CH
ER = 2 * GR
STRIPE = NP // 16
ZR = 120


def _make_degree_kernel():
    DW = 8
    mesh = plsc.VectorSubcoreMesh(core_axis_name="c", subcore_axis_name="s")

    @functools.partial(
        pl.kernel,
        out_type=jax.ShapeDtypeStruct((4 * NP, DW), jnp.float32),
        mesh=mesh,
        compiler_params=pltpu.CompilerParams(use_tc_tiling_on_sc=False),
        scratch_types=[
            pltpu.VMEM_SHARED((NP, DW), jnp.float32),
            pltpu.VMEM((KCH, CH), jnp.int32),
            pltpu.VMEM((CH, DW), jnp.float32),
            pltpu.VMEM((STRIPE, DW), jnp.float32),
            pltpu.SemaphoreType.DMA,
        ],
    )
    def k(dst_hbm, out_hbm, acc, db, ones, zbuf, sem):
        cid = lax.axis_index("c")
        sid = lax.axis_index("s")
        wid = cid * 16 + sid

        def fill_ones(r, carry):
            ones[r, pl.ds(0, DW)] = jnp.ones((DW,), jnp.float32)
            return carry

        def fill_zero(r, carry):
            zbuf[r, pl.ds(0, DW)] = jnp.zeros((DW,), jnp.float32)
            return carry

        lax.fori_loop(0, CH, fill_ones, 0)
        lax.fori_loop(0, STRIPE, fill_zero, 0)

        for g in range(2):
            pltpu.sync_copy(dst_hbm.at[pl.ds(g * GR + wid * KCH, KCH)], db)
            pltpu.sync_copy(zbuf, acc.at[pl.ds(sid * STRIPE, STRIPE)])
            plsc.subcore_barrier()

            def body(t, carry):
                for u in range(4):
                    pltpu.async_copy(ones, acc.at[db.at[t * 4 + u]],
                                     sem, add=True)
                for u in range(4):
                    pltpu.make_async_copy(
                        ones, acc.at[db.at[t * 4 + u]], sem).wait()
                return carry

            lax.fori_loop(0, KCH // 4, body, 0)
            plsc.subcore_barrier()

            pltpu.sync_copy(
                acc.at[pl.ds(sid * STRIPE, STRIPE)],
                out_hbm.at[pl.ds((g * 2 + cid) * NP + sid * STRIPE, STRIPE)])

    return k


def _make_segsum_kernel(F):
    mesh = plsc.VectorSubcoreMesh(core_axis_name="c", subcore_axis_name="s")

    @functools.partial(
        pl.kernel,
        out_type=jax.ShapeDtypeStruct((4 * NP, F), jnp.float32),
        mesh=mesh,
        compiler_params=pltpu.CompilerParams(use_tc_tiling_on_sc=False),
        scratch_types=[
            pltpu.VMEM_SHARED((NP, F), jnp.float32),
            pltpu.VMEM((KCH, CH), jnp.int32),
            pltpu.VMEM((KCH, CH), jnp.int32),
            [pltpu.VMEM((CH, F), jnp.float32) for _ in range(4)],
            pltpu.VMEM((ZR, F), jnp.float32),
            [pltpu.SemaphoreType.DMA for _ in range(4)],
            [pltpu.SemaphoreType.DMA for _ in range(4)],
        ],
    )
    def k(y_hbm, src_hbm, dst_hbm, out_hbm, acc, srcb, dstb, rows, zbuf,
          gsem, ssem):
        cid = lax.axis_index("c")
        sid = lax.axis_index("s")
        wid = cid * 16 + sid

        def fill_zero(r, carry):
            for c in range(F // 16):
                zbuf[r, pl.ds(c * 16, 16)] = jnp.zeros((16,), jnp.float32)
            return carry

        lax.fori_loop(0, ZR, fill_zero, 0)

        for g in range(2):
            pltpu.sync_copy(src_hbm.at[pl.ds(g * GR + wid * KCH, KCH)], srcb)
            pltpu.sync_copy(dst_hbm.at[pl.ds(g * GR + wid * KCH, KCH)], dstb)
            for z in range(STRIPE // ZR):
                pltpu.sync_copy(zbuf,
                                acc.at[pl.ds(sid * STRIPE + z * ZR, ZR)])
            pltpu.sync_copy(zbuf.at[pl.ds(0, STRIPE % ZR)],
                            acc.at[pl.ds(sid * STRIPE + STRIPE // ZR * ZR,
                                         STRIPE % ZR)])
            plsc.subcore_barrier()

            for u in range(4):
                pltpu.async_copy(y_hbm.at[srcb.at[u]], rows[u], gsem[u])

            def body(t, carry):
                k0 = 4 * t
                for u in range(4):
                    pltpu.make_async_copy(y_hbm.at[srcb.at[k0 + u]], rows[u],
                                          gsem[u]).wait()
                    pltpu.async_copy(rows[u], acc.at[dstb.at[k0 + u]],
                                     ssem[u], add=True)
                for u in range(4):
                    pltpu.make_async_copy(rows[u], acc.at[dstb.at[k0 + u]],
                                          ssem[u]).wait()

                    @pl.when(k0 + u + 4 < KCH)
                    def _(u=u, k0=k0):
                        pltpu.async_copy(y_hbm.at[srcb.at[k0 + u + 4]],
                                         rows[u], gsem[u])
                return carry

            lax.fori_loop(0, KCH // 4, body, 0)
            plsc.subcore_barrier()

            pltpu.sync_copy(
                acc.at[pl.ds(sid * STRIPE, STRIPE)],
                out_hbm.at[pl.ds((g * 2 + cid) * NP + sid * STRIPE, STRIPE)])

    return k


def _xw_call(x, W):
    def body(x_ref, w_ref, o_ref):
        o_ref[...] = jnp.dot(x_ref[...], w_ref[...],
                             preferred_element_type=jnp.float32)

    return pl.pallas_call(
        body, out_shape=jax.ShapeDtypeStruct((2 * N, F1), jnp.float32))(x, W)


def _dinv_scale_call(parts, xw):
    def body(p_ref, x_ref, d_ref, y_ref):
        di = lax.rsqrt(1.0 + p_ref[0:NP, 0:1] + p_ref[NP:2 * NP, 0:1])
        dj = lax.rsqrt(1.0 + p_ref[2 * NP:3 * NP, 0:1]
                       + p_ref[3 * NP:4 * NP, 0:1])
        d_ref[0:NP] = di
        d_ref[NP:NT] = dj
        y_ref[0:N] = di[0:N] * x_ref[0:N]
        y_ref[N:NP] = jnp.zeros((NP - N, F1), jnp.float32)
        y_ref[NP:NP + N] = dj[0:N] * x_ref[N:2 * N]
        y_ref[NP + N:NT] = jnp.zeros((NP - N, F1), jnp.float32)

    return pl.pallas_call(
        body,
        out_shape=(jax.ShapeDtypeStruct((NT, 1), jnp.float32),
                   jax.ShapeDtypeStruct((NT, F1), jnp.float32)))(parts, xw)


def _make_mid_call(F, Fn):
    def body(a_ref, y_ref, d_ref, b_ref, w_ref, o_ref):
        s_i = a_ref[0:NP] + a_ref[NP:2 * NP] + y_ref[0:NP]
        s_j = a_ref[2 * NP:3 * NP] + a_ref[3 * NP:4 * NP] + y_ref[NP:NT]
        s = jnp.concatenate([s_i, s_j], axis=0)
        z = jnp.maximum(d_ref[...] * s + b_ref[...], 0.0)
        o_ref[...] = d_ref[...] * jnp.dot(
            z, w_ref[...], preferred_element_type=jnp.float32)
        o_ref[N:NP] = jnp.zeros((NP - N, Fn), jnp.float32)
        o_ref[NP + N:NT] = jnp.zeros((NP - N, Fn), jnp.float32)

    def call(acc, y, dinv, b, W):
        return pl.pallas_call(
            body, out_shape=jax.ShapeDtypeStruct((NT, Fn), jnp.float32))(
                acc, y, dinv, b, W)

    return call


def _att_call(acc, y, dinv, b, W_att):
    def body(a_ref, y_ref, d_ref, b_ref, w_ref, o_ref):
        s_i = a_ref[0:NP] + a_ref[NP:2 * NP] + y_ref[0:NP]
        s_j = a_ref[2 * NP:3 * NP] + a_ref[3 * NP:4 * NP] + y_ref[NP:NT]
        enc = d_ref[...] * jnp.concatenate([s_i, s_j], axis=0) + b_ref[...]
        for g in range(2):
            encn = enc[g * NP:g * NP + N]
            m = jnp.sum(encn, axis=0, keepdims=True) * (1.0 / N)
            c = jnp.tanh(jnp.dot(m, w_ref[...],
                                 preferred_element_type=jnp.float32))
            s = jax.nn.sigmoid(jnp.sum(encn * c, axis=1, keepdims=True))
            o_ref[g:g + 1, :] = jnp.sum(encn * s, axis=0, keepdims=True)

    return pl.pallas_call(
        body, out_shape=jax.ShapeDtypeStruct((2, F3), jnp.float32))(
            acc, y, dinv, b, W_att)


def _head_call(h, W_ntn, V_ntn_t, b_ntn, W_m0, b_m0, W_m1, b_m1,
               W_m2, b_m2, W_m3, b_m3, W_s, b_s):
    def body(h_ref, wn_ref, vt_ref, bn_ref, w0_ref, c0_ref, w1_ref,
             c1_ref, w2_ref, c2_ref, w3_ref, c3_ref, ws_ref, cs_ref, o_ref):
        hiv = h_ref[0:1]
        hjv = h_ref[1:2]
        ks = lax.broadcasted_iota(jnp.int32, (1, NTN_SLICES), 1)
        bil = jnp.zeros((1, NTN_SLICES), jnp.float32)
        for k in range(NTN_SLICES):
            wk = wn_ref[k]
            val = jnp.sum(jnp.dot(hiv, wk, preferred_element_type=jnp.float32)
                          * hjv)
            bil = bil + jnp.where(ks == k, val, 0.0)
        cat = jnp.concatenate([hiv, hjv], axis=1)
        lin = jnp.dot(cat, vt_ref[...], preferred_element_type=jnp.float32)
        inter = jnp.tanh(bil + lin + bn_ref[...])
        sim = jax.nn.sigmoid(jnp.sum(hiv * hjv))
        binx = jnp.clip((sim * HIST_BINS).astype(jnp.int32), 0, HIST_BINS - 1)
        hist = jnp.where(ks == binx, 1.0, 0.0)
        feat = jnp.concatenate([inter, hist], axis=1)
        for w_ref, c_ref in ((w0_ref, c0_ref), (w1_ref, c1_ref),
                             (w2_ref, c2_ref), (w3_ref, c3_ref)):
            feat = jnp.maximum(
                jnp.dot(feat, w_ref[...], preferred_element_type=jnp.float32)
                + c_ref[...], 0.0)
        o_ref[...] = jnp.dot(feat, ws_ref[...],
                             preferred_element_type=jnp.float32) + cs_ref[...]

    return pl.pallas_call(
        body, out_shape=jax.ShapeDtypeStruct((1, 1), jnp.float32))(
            h, W_ntn, V_ntn_t, b_ntn, W_m0, b_m0, W_m1, b_m1,
            W_m2, b_m2, W_m3, b_m3, W_s, b_s)


_deg_kernel = _make_degree_kernel()
_seg64 = _make_segsum_kernel(F1)
_seg32 = _make_segsum_kernel(F2)
_seg16 = _make_segsum_kernel(F3)
_mid_64_32 = _make_mid_call(F1, F2)
_mid_32_16 = _make_mid_call(F2, F3)


def _pad_edges(e, base):
    fill = N + jnp.arange(EP - E, dtype=jnp.int32) % (NP - N)
    return jnp.concatenate([e + base, fill + base])


def kernel(x_i, x_j, edge_index_i, edge_index_j, W1, b1, W2, b2, W3, b3,
           W_att, W_ntn, V_ntn, b_ntn, W_m0, b_m0, W_m1, b_m1, W_m2, b_m2,
           W_m3, b_m3, W_s, b_s):
    src = jnp.concatenate([_pad_edges(edge_index_i[0], 0),
                           _pad_edges(edge_index_j[0], NP)]).reshape(ER, CH)
    dst = jnp.concatenate([_pad_edges(edge_index_i[1], 0),
                           _pad_edges(edge_index_j[1], 0)]).reshape(ER, CH)

    deg_parts = _deg_kernel(dst)
    xw1 = _xw_call(jnp.concatenate([x_i, x_j], axis=0), W1)
    dinv, y1 = _dinv_scale_call(deg_parts, xw1)

    a1 = _seg64(y1, src, dst)
    y2 = _mid_64_32(a1, y1, dinv, b1.reshape(1, F1), W2)
    a2 = _seg32(y2, src, dst)
    y3 = _mid_32_16(a2, y2, dinv, b2.reshape(1, F2), W3)
    a3 = _seg16(y3, src, dst)
    h = _att_call(a3, y3, dinv, b3.reshape(1, F3), W_att)

    out = _head_call(h, W_ntn, V_ntn.T, b_ntn.reshape(1, NTN_SLICES),
                     W_m0, b_m0.reshape(1, 32), W_m1, b_m1.reshape(1, 16),
                     W_m2, b_m2.reshape(1, 8), W_m3, b_m3.reshape(1, 4),
                     W_s, b_s.reshape(1, 1))
    return out.reshape(1)


# speedup vs baseline: 39.9610x; 1.0095x over previous
import functools

import jax
import jax.numpy as jnp
from jax import lax
from jax.experimental import pallas as pl
from jax.experimental.pallas import tpu as pltpu
from jax.experimental.pallas import tpu_sc as plsc

N = 10000
NP = 10112
NT = 2 * NP
E = 320000
D_IN = 128
F1, F2, F3 = 64, 32, 16
NTN_SLICES = 16
HIST_BINS = 16

NWORK = 32
CH = 128
KCH = 80
EP = NWORK * KCH * CH
GR = EP // CH
ER = 2 * GR
STRIPE = NP // 16
ZR = 120


def _make_degree_kernel():
    DW = 8
    mesh = plsc.VectorSubcoreMesh(core_axis_name="c", subcore_axis_name="s")

    @functools.partial(
        pl.kernel,
        out_type=jax.ShapeDtypeStruct((4 * NP, DW), jnp.float32),
        mesh=mesh,
        compiler_params=pltpu.CompilerParams(use_tc_tiling_on_sc=False),
        scratch_types=[
            pltpu.VMEM_SHARED((NP, DW), jnp.float32),
            pltpu.VMEM((KCH, CH), jnp.int32),
            pltpu.VMEM((CH, DW), jnp.float32),
            pltpu.VMEM((STRIPE, DW), jnp.float32),
            pltpu.SemaphoreType.DMA,
        ],
    )
    def k(dst_hbm, out_hbm, acc, db, ones, zbuf, sem):
        cid = lax.axis_index("c")
        sid = lax.axis_index("s")
        wid = cid * 16 + sid

        def fill_ones(r, carry):
            ones[r, pl.ds(0, DW)] = jnp.ones((DW,), jnp.float32)
            return carry

        def fill_zero(r, carry):
            zbuf[r, pl.ds(0, DW)] = jnp.zeros((DW,), jnp.float32)
            return carry

        lax.fori_loop(0, CH, fill_ones, 0)
        lax.fori_loop(0, STRIPE, fill_zero, 0)

        for g in range(2):
            pltpu.sync_copy(dst_hbm.at[pl.ds(g * GR + wid * KCH, KCH)], db)
            pltpu.sync_copy(zbuf, acc.at[pl.ds(sid * STRIPE, STRIPE)])
            plsc.subcore_barrier()

            def body(t, carry):
                for u in range(4):
                    pltpu.async_copy(ones, acc.at[db.at[t * 4 + u]],
                                     sem, add=True)
                for u in range(4):
                    pltpu.make_async_copy(
                        ones, acc.at[db.at[t * 4 + u]], sem).wait()
                return carry

            lax.fori_loop(0, KCH // 4, body, 0)
            plsc.subcore_barrier()

            pltpu.sync_copy(
                acc.at[pl.ds(sid * STRIPE, STRIPE)],
                out_hbm.at[pl.ds((g * 2 + cid) * NP + sid * STRIPE, STRIPE)])

    return k


def _make_segsum_kernel(F):
    mesh = plsc.VectorSubcoreMesh(core_axis_name="c", subcore_axis_name="s")

    @functools.partial(
        pl.kernel,
        out_type=jax.ShapeDtypeStruct((4 * NP, F), jnp.float32),
        mesh=mesh,
        compiler_params=pltpu.CompilerParams(use_tc_tiling_on_sc=False),
        scratch_types=[
            pltpu.VMEM_SHARED((NP, F), jnp.float32),
            pltpu.VMEM((KCH, CH), jnp.int32),
            pltpu.VMEM((KCH, CH), jnp.int32),
            [pltpu.VMEM((CH, F), jnp.float32) for _ in range(4)],
            pltpu.VMEM((ZR, F), jnp.float32),
            [pltpu.SemaphoreType.DMA for _ in range(4)],
            [pltpu.SemaphoreType.DMA for _ in range(4)],
        ],
    )
    def k(yi_hbm, yj_hbm, src_hbm, dst_hbm, out_hbm, acc, srcb, dstb, rows,
          zbuf, gsem, ssem):
        cid = lax.axis_index("c")
        sid = lax.axis_index("s")
        wid = cid * 16 + sid

        def fill_zero(r, carry):
            for c in range(F // 16):
                zbuf[r, pl.ds(c * 16, 16)] = jnp.zeros((16,), jnp.float32)
            return carry

        lax.fori_loop(0, ZR, fill_zero, 0)

        for g, y_hbm in enumerate((yi_hbm, yj_hbm)):
            pltpu.sync_copy(src_hbm.at[pl.ds(g * GR + wid * KCH, KCH)], srcb)
            pltpu.sync_copy(dst_hbm.at[pl.ds(g * GR + wid * KCH, KCH)], dstb)
            for z in range(STRIPE // ZR):
                pltpu.sync_copy(zbuf,
                                acc.at[pl.ds(sid * STRIPE + z * ZR, ZR)])
            pltpu.sync_copy(zbuf.at[pl.ds(0, STRIPE % ZR)],
                            acc.at[pl.ds(sid * STRIPE + STRIPE // ZR * ZR,
                                         STRIPE % ZR)])
            plsc.subcore_barrier()

            for u in range(4):
                pltpu.async_copy(y_hbm.at[srcb.at[u]], rows[u], gsem[u])

            def body(t, carry, y_hbm=y_hbm):
                k0 = 4 * t
                for u in range(4):
                    pltpu.make_async_copy(y_hbm.at[srcb.at[k0 + u]], rows[u],
                                          gsem[u]).wait()
                    pltpu.async_copy(rows[u], acc.at[dstb.at[k0 + u]],
                                     ssem[u], add=True)
                for u in range(4):
                    pltpu.make_async_copy(rows[u], acc.at[dstb.at[k0 + u]],
                                          ssem[u]).wait()

                    @pl.when(k0 + u + 4 < KCH)
                    def _(u=u, k0=k0, y_hbm=y_hbm):
                        pltpu.async_copy(y_hbm.at[srcb.at[k0 + u + 4]],
                                         rows[u], gsem[u])
                return carry

            lax.fori_loop(0, KCH // 4, body, 0)
            plsc.subcore_barrier()

            pltpu.sync_copy(
                acc.at[pl.ds(sid * STRIPE, STRIPE)],
                out_hbm.at[pl.ds((g * 2 + cid) * NP + sid * STRIPE, STRIPE)])

    return k


def _dinv_scale_call(parts, x_i, x_j, W1):
    def body(p_ref, xi_ref, xj_ref, w_ref, d_ref, yi_ref, yj_ref):
        di = lax.rsqrt(1.0 + p_ref[0:NP, 0:1] + p_ref[NP:2 * NP, 0:1])
        dj = lax.rsqrt(1.0 + p_ref[2 * NP:3 * NP, 0:1]
                       + p_ref[3 * NP:4 * NP, 0:1])
        d_ref[0:NP] = di
        d_ref[NP:NT] = dj
        yi_ref[0:N] = di[0:N] * jnp.dot(xi_ref[...], w_ref[...],
                                        preferred_element_type=jnp.float32)
        yi_ref[N:NP] = jnp.zeros((NP - N, F1), jnp.float32)
        yj_ref[0:N] = dj[0:N] * jnp.dot(xj_ref[...], w_ref[...],
                                        preferred_element_type=jnp.float32)
        yj_ref[N:NP] = jnp.zeros((NP - N, F1), jnp.float32)

    return pl.pallas_call(
        body,
        out_shape=(jax.ShapeDtypeStruct((NT, 1), jnp.float32),
                   jax.ShapeDtypeStruct((NP, F1), jnp.float32),
                   jax.ShapeDtypeStruct((NP, F1), jnp.float32)))(
                       parts, x_i, x_j, W1)


def _make_mid_call(F, Fn):
    def body(a_ref, yi_ref, yj_ref, d_ref, b_ref, w_ref, oi_ref, oj_ref):
        for g, (y_ref, o_ref) in enumerate(((yi_ref, oi_ref),
                                            (yj_ref, oj_ref))):
            s = (a_ref[2 * g * NP:(2 * g + 1) * NP]
                 + a_ref[(2 * g + 1) * NP:(2 * g + 2) * NP] + y_ref[...])
            d = d_ref[g * NP:(g + 1) * NP]
            z = jnp.maximum(d * s + b_ref[...], 0.0)
            o_ref[0:N] = d[0:N] * jnp.dot(
                z[0:N], w_ref[...], preferred_element_type=jnp.float32)
            o_ref[N:NP] = jnp.zeros((NP - N, Fn), jnp.float32)

    def call(acc, y_i, y_j, dinv, b, W):
        return pl.pallas_call(
            body, out_shape=(jax.ShapeDtypeStruct((NP, Fn), jnp.float32),
                             jax.ShapeDtypeStruct((NP, Fn), jnp.float32)))(
                acc, y_i, y_j, dinv, b, W)

    return call


def _att_head_call(acc, y_i, y_j, dinv, b3r, W_att, W_ntn, V_ntn_t, b_ntn,
                   W_m0, b_m0, W_m1, b_m1, W_m2, b_m2, W_m3, b_m3, W_s, b_s):
    def body(a_ref, yi_ref, yj_ref, d_ref, b_ref, w_ref, wn_ref, vt_ref,
             bn_ref, w0_ref, c0_ref, w1_ref, c1_ref, w2_ref, c2_ref, w3_ref,
             c3_ref, ws_ref, cs_ref, o_ref):
        hs = []
        for g, y_ref in enumerate((yi_ref, yj_ref)):
            s_g = (a_ref[2 * g * NP:(2 * g + 1) * NP]
                   + a_ref[(2 * g + 1) * NP:(2 * g + 2) * NP] + y_ref[...])
            enc = d_ref[g * NP:(g + 1) * NP] * s_g + b_ref[...]
            encn = enc[0:N]
            m = jnp.sum(encn, axis=0, keepdims=True) * (1.0 / N)
            c = jnp.tanh(jnp.dot(m, w_ref[...],
                                 preferred_element_type=jnp.float32))
            s = jax.nn.sigmoid(jnp.sum(encn * c, axis=1, keepdims=True))
            hs.append(jnp.sum(encn * s, axis=0, keepdims=True))
        hiv, hjv = hs
        ks = lax.broadcasted_iota(jnp.int32, (1, NTN_SLICES), 1)
        bil = jnp.zeros((1, NTN_SLICES), jnp.float32)
        for k in range(NTN_SLICES):
            wk = wn_ref[k]
            val = jnp.sum(jnp.dot(hiv, wk, preferred_element_type=jnp.float32)
                          * hjv)
            bil = bil + jnp.where(ks == k, val, 0.0)
        cat = jnp.concatenate([hiv, hjv], axis=1)
        lin = jnp.dot(cat, vt_ref[...], preferred_element_type=jnp.float32)
        inter = jnp.tanh(bil + lin + bn_ref[...])
        sim = jax.nn.sigmoid(jnp.sum(hiv * hjv))
        binx = jnp.clip((sim * HIST_BINS).astype(jnp.int32), 0, HIST_BINS - 1)
        hist = jnp.where(ks == binx, 1.0, 0.0)
        feat = jnp.concatenate([inter, hist], axis=1)
        for w_ref, c_ref in ((w0_ref, c0_ref), (w1_ref, c1_ref),
                             (w2_ref, c2_ref), (w3_ref, c3_ref)):
            feat = jnp.maximum(
                jnp.dot(feat, w_ref[...], preferred_element_type=jnp.float32)
                + c_ref[...], 0.0)
        o_ref[...] = jnp.dot(feat, ws_ref[...],
                             preferred_element_type=jnp.float32) + cs_ref[...]

    return pl.pallas_call(
        body, out_shape=jax.ShapeDtypeStruct((1, 1), jnp.float32))(
            acc, y_i, y_j, dinv, b3r, W_att, W_ntn, V_ntn_t, b_ntn,
            W_m0, b_m0, W_m1, b_m1, W_m2, b_m2, W_m3, b_m3, W_s, b_s)


_deg_kernel = _make_degree_kernel()
_seg64 = _make_segsum_kernel(F1)
_seg32 = _make_segsum_kernel(F2)
_seg16 = _make_segsum_kernel(F3)
_mid_64_32 = _make_mid_call(F1, F2)
_mid_32_16 = _make_mid_call(F2, F3)


def _pad_edges(e):
    fill = N + jnp.arange(EP - E, dtype=jnp.int32) % (NP - N)
    return jnp.concatenate([e, fill])


def kernel(x_i, x_j, edge_index_i, edge_index_j, W1, b1, W2, b2, W3, b3,
           W_att, W_ntn, V_ntn, b_ntn, W_m0, b_m0, W_m1, b_m1, W_m2, b_m2,
           W_m3, b_m3, W_s, b_s):
    src = jnp.concatenate([_pad_edges(edge_index_i[0]),
                           _pad_edges(edge_index_j[0])]).reshape(ER, CH)
    dst = jnp.concatenate([_pad_edges(edge_index_i[1]),
                           _pad_edges(edge_index_j[1])]).reshape(ER, CH)

    deg_parts = _deg_kernel(dst)
    dinv, y1_i, y1_j = _dinv_scale_call(deg_parts, x_i, x_j, W1)

    a1 = _seg64(y1_i, y1_j, src, dst)
    y2_i, y2_j = _mid_64_32(a1, y1_i, y1_j, dinv, b1.reshape(1, F1), W2)
    a2 = _seg32(y2_i, y2_j, src, dst)
    y3_i, y3_j = _mid_32_16(a2, y2_i, y2_j, dinv, b2.reshape(1, F2), W3)
    a3 = _seg16(y3_i, y3_j, src, dst)

    out = _att_head_call(a3, y3_i, y3_j, dinv, b3.reshape(1, F3), W_att,
                         W_ntn, V_ntn.T, b_ntn.reshape(1, NTN_SLICES),
                         W_m0, b_m0.reshape(1, 32), W_m1, b_m1.reshape(1, 16),
                         W_m2, b_m2.reshape(1, 8), W_m3, b_m3.reshape(1, 4),
                         W_s, b_s.reshape(1, 1))
    return out.reshape(1)


# speedup vs baseline: 41.1852x; 1.0306x over previous
import functools

import jax
import jax.numpy as jnp
from jax import lax
from jax.experimental import pallas as pl
from jax.experimental.pallas import tpu as pltpu
from jax.experimental.pallas import tpu_sc as plsc

N = 10000
NP = 10112
NT = 2 * NP
E = 320000
D_IN = 128
F1, F2, F3 = 64, 32, 16
NTN_SLICES = 16
HIST_BINS = 16

NWORK = 32
CH = 128
KCH = 80
EP = NWORK * KCH * CH
GR = EP // CH
ER = 2 * GR
STRIPE = NP // 16
ZR = 120


def _make_degree_kernel():
    DW = 8
    mesh = plsc.VectorSubcoreMesh(core_axis_name="c", subcore_axis_name="s")

    @functools.partial(
        pl.kernel,
        out_type=jax.ShapeDtypeStruct((NP, 4 * DW), jnp.float32),
        mesh=mesh,
        compiler_params=pltpu.CompilerParams(use_tc_tiling_on_sc=False),
        scratch_types=[
            pltpu.VMEM_SHARED((NP, DW), jnp.float32),
            pltpu.VMEM((KCH, CH), jnp.int32),
            pltpu.VMEM((CH, DW), jnp.float32),
            pltpu.VMEM((STRIPE, DW), jnp.float32),
            pltpu.SemaphoreType.DMA,
        ],
    )
    def k(dst_hbm, out_hbm, acc, db, ones, zbuf, sem):
        cid = lax.axis_index("c")
        sid = lax.axis_index("s")
        wid = cid * 16 + sid

        def fill_ones(r, carry):
            ones[r, pl.ds(0, DW)] = jnp.ones((DW,), jnp.float32)
            return carry

        def fill_zero(r, carry):
            zbuf[r, pl.ds(0, DW)] = jnp.zeros((DW,), jnp.float32)
            return carry

        lax.fori_loop(0, CH, fill_ones, 0)
        lax.fori_loop(0, STRIPE, fill_zero, 0)

        for g in range(2):
            pltpu.sync_copy(dst_hbm.at[pl.ds(g * GR + wid * KCH, KCH)], db)
            pltpu.sync_copy(zbuf, acc.at[pl.ds(sid * STRIPE, STRIPE)])
            plsc.subcore_barrier()

            def body(t, carry):
                for u in range(4):
                    pltpu.async_copy(ones, acc.at[db.at[t * 4 + u]],
                                     sem, add=True)
                for u in range(4):
                    pltpu.make_async_copy(
                        ones, acc.at[db.at[t * 4 + u]], sem).wait()
                return carry

            lax.fori_loop(0, KCH // 4, body, 0)
            plsc.subcore_barrier()

            pltpu.sync_copy(
                acc.at[pl.ds(sid * STRIPE, STRIPE)],
                out_hbm.at[pl.ds(sid * STRIPE, STRIPE),
                           pl.ds((g * 2 + cid) * DW, DW)])

    return k


def _make_segsum_kernel(F):
    mesh = plsc.VectorSubcoreMesh(core_axis_name="c", subcore_axis_name="s")

    @functools.partial(
        pl.kernel,
        out_type=jax.ShapeDtypeStruct((NP, 4 * F), jnp.float32),
        mesh=mesh,
        compiler_params=pltpu.CompilerParams(use_tc_tiling_on_sc=False),
        scratch_types=[
            pltpu.VMEM_SHARED((NP, F), jnp.float32),
            pltpu.VMEM((KCH, CH), jnp.int32),
            pltpu.VMEM((KCH, CH), jnp.int32),
            [pltpu.VMEM((CH, F), jnp.float32) for _ in range(4)],
            pltpu.VMEM((ZR, F), jnp.float32),
            [pltpu.SemaphoreType.DMA for _ in range(4)],
            [pltpu.SemaphoreType.DMA for _ in range(4)],
        ],
    )
    def k(yi_hbm, yj_hbm, src_hbm, dst_hbm, out_hbm, acc, srcb, dstb, rows,
          zbuf, gsem, ssem):
        cid = lax.axis_index("c")
        sid = lax.axis_index("s")
        wid = cid * 16 + sid

        def fill_zero(r, carry):
            for c in range(F // 16):
                zbuf[r, pl.ds(c * 16, 16)] = jnp.zeros((16,), jnp.float32)
            return carry

        lax.fori_loop(0, ZR, fill_zero, 0)

        for g, y_hbm in enumerate((yi_hbm, yj_hbm)):
            pltpu.sync_copy(src_hbm.at[pl.ds(g * GR + wid * KCH, KCH)], srcb)
            pltpu.sync_copy(dst_hbm.at[pl.ds(g * GR + wid * KCH, KCH)], dstb)
            for z in range(STRIPE // ZR):
                pltpu.sync_copy(zbuf,
                                acc.at[pl.ds(sid * STRIPE + z * ZR, ZR)])
            pltpu.sync_copy(zbuf.at[pl.ds(0, STRIPE % ZR)],
                            acc.at[pl.ds(sid * STRIPE + STRIPE // ZR * ZR,
                                         STRIPE % ZR)])
            plsc.subcore_barrier()

            for u in range(4):
                pltpu.async_copy(y_hbm.at[srcb.at[u]], rows[u], gsem[u])

            def body(t, carry, y_hbm=y_hbm):
                k0 = 4 * t
                for u in range(4):
                    pltpu.make_async_copy(y_hbm.at[srcb.at[k0 + u]], rows[u],
                                          gsem[u]).wait()
                    pltpu.async_copy(rows[u], acc.at[dstb.at[k0 + u]],
                                     ssem[u], add=True)
                for u in range(4):
                    pltpu.make_async_copy(rows[u], acc.at[dstb.at[k0 + u]],
                                          ssem[u]).wait()

                    @pl.when(k0 + u + 4 < KCH)
                    def _(u=u, k0=k0, y_hbm=y_hbm):
                        pltpu.async_copy(y_hbm.at[srcb.at[k0 + u + 4]],
                                         rows[u], gsem[u])
                return carry

            lax.fori_loop(0, KCH // 4, body, 0)
            plsc.subcore_barrier()

            pltpu.sync_copy(
                acc.at[pl.ds(sid * STRIPE, STRIPE)],
                out_hbm.at[pl.ds(sid * STRIPE, STRIPE),
                           pl.ds((g * 2 + cid) * F, F)])

    return k


def _dinv_scale_call(parts, x_i, x_j, W1):
    def body(p_ref, xi_ref, xj_ref, w_ref, d_ref, yi_ref, yj_ref):
        di = lax.rsqrt(1.0 + p_ref[:, 0:1] + p_ref[:, 8:9])
        dj = lax.rsqrt(1.0 + p_ref[:, 16:17] + p_ref[:, 24:25])
        d_ref[0:NP] = di
        d_ref[NP:NT] = dj
        for d, x_ref, y_ref in ((di, xi_ref, yi_ref), (dj, xj_ref, yj_ref)):
            y_ref[0:N] = d[0:N] * jnp.dot(
                x_ref[...], w_ref[...], preferred_element_type=jnp.float32)
            y_ref[N:NP] = jnp.zeros((NP - N, F1), jnp.float32)

    return pl.pallas_call(
        body,
        out_shape=(jax.ShapeDtypeStruct((NT, 1), jnp.float32),
                   jax.ShapeDtypeStruct((NP, F1), jnp.float32),
                   jax.ShapeDtypeStruct((NP, F1), jnp.float32)))(
                       parts, x_i, x_j, W1)


def _make_mid_call(F, Fn):
    def body(a_ref, yi_ref, yj_ref, d_ref, b_ref, w_ref, oi_ref, oj_ref):
        for g, (y_ref, o_ref) in enumerate(((yi_ref, oi_ref),
                                            (yj_ref, oj_ref))):
            s = (a_ref[:, 2 * g * F:(2 * g + 1) * F]
                 + a_ref[:, (2 * g + 1) * F:(2 * g + 2) * F] + y_ref[...])
            d = d_ref[g * NP:(g + 1) * NP]
            z = jnp.maximum(d * s + b_ref[...], 0.0)
            o_ref[0:N] = d[0:N] * jnp.dot(
                z[0:N], w_ref[...], preferred_element_type=jnp.float32)
            o_ref[N:NP] = jnp.zeros((NP - N, Fn), jnp.float32)

    def call(acc, y_i, y_j, dinv, b, W):
        return pl.pallas_call(
            body, out_shape=(jax.ShapeDtypeStruct((NP, Fn), jnp.float32),
                             jax.ShapeDtypeStruct((NP, Fn), jnp.float32)))(
                acc, y_i, y_j, dinv, b, W)

    return call


def _att_head_call(acc, y_i, y_j, dinv, b3r, W_att, W_ntn, V_ntn_t, b_ntn,
                   W_m0, b_m0, W_m1, b_m1, W_m2, b_m2, W_m3, b_m3, W_s, b_s):
    def body(a_ref, yi_ref, yj_ref, d_ref, b_ref, w_ref, wn_ref, vt_ref,
             bn_ref, w0_ref, c0_ref, w1_ref, c1_ref, w2_ref, c2_ref, w3_ref,
             c3_ref, ws_ref, cs_ref, o_ref):
        hs = []
        for g, y_ref in enumerate((yi_ref, yj_ref)):
            s_g = (a_ref[:, 2 * g * F3:(2 * g + 1) * F3]
                   + a_ref[:, (2 * g + 1) * F3:(2 * g + 2) * F3]
                   + y_ref[...])
            enc = d_ref[g * NP:(g + 1) * NP] * s_g + b_ref[...]
            encn = enc[0:N]
            m = jnp.sum(encn, axis=0, keepdims=True) * (1.0 / N)
            c = jnp.tanh(jnp.dot(m, w_ref[...],
                                 preferred_element_type=jnp.float32))
            s = jax.nn.sigmoid(jnp.sum(encn * c, axis=1, keepdims=True))
            hs.append(jnp.sum(encn * s, axis=0, keepdims=True))
        hiv, hjv = hs
        ks = lax.broadcasted_iota(jnp.int32, (1, NTN_SLICES), 1)
        bil = jnp.zeros((1, NTN_SLICES), jnp.float32)
        for k in range(NTN_SLICES):
            wk = wn_ref[k]
            val = jnp.sum(jnp.dot(hiv, wk, preferred_element_type=jnp.float32)
                          * hjv)
            bil = bil + jnp.where(ks == k, val, 0.0)
        cat = jnp.concatenate([hiv, hjv], axis=1)
        lin = jnp.dot(cat, vt_ref[...], preferred_element_type=jnp.float32)
        inter = jnp.tanh(bil + lin + bn_ref[...])
        sim = jax.nn.sigmoid(jnp.sum(hiv * hjv))
        binx = jnp.clip((sim * HIST_BINS).astype(jnp.int32), 0, HIST_BINS - 1)
        hist = jnp.where(ks == binx, 1.0, 0.0)
        feat = jnp.concatenate([inter, hist], axis=1)
        for w_ref, c_ref in ((w0_ref, c0_ref), (w1_ref, c1_ref),
                             (w2_ref, c2_ref), (w3_ref, c3_ref)):
            feat = jnp.maximum(
                jnp.dot(feat, w_ref[...], preferred_element_type=jnp.float32)
                + c_ref[...], 0.0)
        o_ref[...] = jnp.dot(feat, ws_ref[...],
                             preferred_element_type=jnp.float32) + cs_ref[...]

    return pl.pallas_call(
        body, out_shape=jax.ShapeDtypeStruct((1, 1), jnp.float32))(
            acc, y_i, y_j, dinv, b3r, W_att, W_ntn, V_ntn_t, b_ntn,
            W_m0, b_m0, W_m1, b_m1, W_m2, b_m2, W_m3, b_m3, W_s, b_s)


_deg_kernel = _make_degree_kernel()
_seg64 = _make_segsum_kernel(F1)
_seg32 = _make_segsum_kernel(F2)
_seg16 = _make_segsum_kernel(F3)
_mid_64_32 = _make_mid_call(F1, F2)
_mid_32_16 = _make_mid_call(F2, F3)


def _pad_edges(e):
    fill = N + jnp.arange(EP - E, dtype=jnp.int32) % (NP - N)
    return jnp.concatenate([e, fill])


def kernel(x_i, x_j, edge_index_i, edge_index_j, W1, b1, W2, b2, W3, b3,
           W_att, W_ntn, V_ntn, b_ntn, W_m0, b_m0, W_m1, b_m1, W_m2, b_m2,
           W_m3, b_m3, W_s, b_s):
    src = jnp.concatenate([_pad_edges(edge_index_i[0]),
                           _pad_edges(edge_index_j[0])]).reshape(ER, CH)
    dst = jnp.concatenate([_pad_edges(edge_index_i[1]),
                           _pad_edges(edge_index_j[1])]).reshape(ER, CH)

    deg_parts = _deg_kernel(dst)
    dinv, y1_i, y1_j = _dinv_scale_call(deg_parts, x_i, x_j, W1)

    a1 = _seg64(y1_i, y1_j, src, dst)
    y2_i, y2_j = _mid_64_32(a1, y1_i, y1_j, dinv, b1.reshape(1, F1), W2)
    a2 = _seg32(y2_i, y2_j, src, dst)
    y3_i, y3_j = _mid_32_16(a2, y2_i, y2_j, dinv, b2.reshape(1, F2), W3)
    a3 = _seg16(y3_i, y3_j, src, dst)

    out = _att_head_call(a3, y3_i, y3_j, dinv, b3.reshape(1, F3), W_att,
                         W_ntn, V_ntn.T, b_ntn.reshape(1, NTN_SLICES),
                         W_m0, b_m0.reshape(1, 32), W_m1, b_m1.reshape(1, 16),
                         W_m2, b_m2.reshape(1, 8), W_m3, b_m3.reshape(1, 4),
                         W_s, b_s.reshape(1, 1))
    return out.reshape(1)


# speedup vs baseline: 43.0131x; 1.0444x over previous
import functools

import jax
import jax.numpy as jnp
from jax import lax
from jax.experimental import pallas as pl
from jax.experimental.pallas import tpu as pltpu
from jax.experimental.pallas import tpu_sc as plsc

N = 10000
NP = 10112
NT = 2 * NP
E = 320000
D_IN = 128
F1, F2, F3 = 64, 32, 16
NTN_SLICES = 16
HIST_BINS = 16

NWORK = 32
CH = 128
KCH = 80
EP = NWORK * KCH * CH
GR = EP // CH
ER = 2 * GR
STRIPE = NP // 16
ZR = 120


def _make_degree_kernel():
    DW = 8
    mesh = plsc.VectorSubcoreMesh(core_axis_name="c", subcore_axis_name="s")

    @functools.partial(
        pl.kernel,
        out_type=jax.ShapeDtypeStruct((NP, 4 * DW), jnp.float32),
        mesh=mesh,
        compiler_params=pltpu.CompilerParams(use_tc_tiling_on_sc=False),
        scratch_types=[
            pltpu.VMEM_SHARED((NP, DW), jnp.float32),
            pltpu.VMEM((KCH, CH), jnp.int32),
            pltpu.VMEM((CH, DW), jnp.float32),
            pltpu.VMEM((STRIPE, DW), jnp.float32),
            pltpu.SemaphoreType.DMA,
        ],
    )
    def k(dst_hbm, out_hbm, acc, db, ones, zbuf, sem):
        cid = lax.axis_index("c")
        sid = lax.axis_index("s")
        wid = cid * 16 + sid

        def fill_ones(r, carry):
            ones[r, pl.ds(0, DW)] = jnp.ones((DW,), jnp.float32)
            return carry

        def fill_zero(r, carry):
            zbuf[r, pl.ds(0, DW)] = jnp.zeros((DW,), jnp.float32)
            return carry

        lax.fori_loop(0, CH, fill_ones, 0)
        lax.fori_loop(0, STRIPE, fill_zero, 0)

        for g in range(2):
            pltpu.sync_copy(dst_hbm.at[pl.ds(g * GR + wid * KCH, KCH)], db)
            pltpu.sync_copy(zbuf, acc.at[pl.ds(sid * STRIPE, STRIPE)])
            plsc.subcore_barrier()

            def body(t, carry):
                for u in range(4):
                    pltpu.async_copy(ones, acc.at[db.at[t * 4 + u]],
                                     sem, add=True)
                for u in range(4):
                    pltpu.make_async_copy(
                        ones, acc.at[db.at[t * 4 + u]], sem).wait()
                return carry

            lax.fori_loop(0, KCH // 4, body, 0)
            plsc.subcore_barrier()

            pltpu.sync_copy(
                acc.at[pl.ds(sid * STRIPE, STRIPE)],
                out_hbm.at[pl.ds(sid * STRIPE, STRIPE),
                           pl.ds((g * 2 + cid) * DW, DW)])

    return k


def _make_segsum_kernel(F):
    split = F == 64
    if split:
        out_type = (jax.ShapeDtypeStruct((NP, 2 * F), jnp.float32),
                    jax.ShapeDtypeStruct((NP, 2 * F), jnp.float32))
    else:
        out_type = jax.ShapeDtypeStruct((NP, 4 * F), jnp.float32)
    mesh = plsc.VectorSubcoreMesh(core_axis_name="c", subcore_axis_name="s")

    @functools.partial(
        pl.kernel,
        out_type=out_type,
        mesh=mesh,
        compiler_params=pltpu.CompilerParams(use_tc_tiling_on_sc=False),
        scratch_types=[
            pltpu.VMEM_SHARED((NP, F), jnp.float32),
            pltpu.VMEM((KCH, CH), jnp.int32),
            pltpu.VMEM((KCH, CH), jnp.int32),
            [pltpu.VMEM((CH, F), jnp.float32) for _ in range(4)],
            pltpu.VMEM((ZR, F), jnp.float32),
            [pltpu.SemaphoreType.DMA for _ in range(4)],
            [pltpu.SemaphoreType.DMA for _ in range(4)],
        ],
    )
    def k(yi_hbm, yj_hbm, src_hbm, dst_hbm, *out_scratch):
        if split:
            (outi_hbm, outj_hbm, acc, srcb, dstb, rows, zbuf, gsem,
             ssem) = out_scratch
            outs = (outi_hbm, outj_hbm)
        else:
            out_hbm, acc, srcb, dstb, rows, zbuf, gsem, ssem = out_scratch
        cid = lax.axis_index("c")
        sid = lax.axis_index("s")
        wid = cid * 16 + sid

        def fill_zero(r, carry):
            for c in range(F // 16):
                zbuf[r, pl.ds(c * 16, 16)] = jnp.zeros((16,), jnp.float32)
            return carry

        lax.fori_loop(0, ZR, fill_zero, 0)

        for g, y_hbm in enumerate((yi_hbm, yj_hbm)):
            pltpu.sync_copy(src_hbm.at[pl.ds(g * GR + wid * KCH, KCH)], srcb)
            pltpu.sync_copy(dst_hbm.at[pl.ds(g * GR + wid * KCH, KCH)], dstb)
            for z in range(STRIPE // ZR):
                pltpu.sync_copy(zbuf,
                                acc.at[pl.ds(sid * STRIPE + z * ZR, ZR)])
            pltpu.sync_copy(zbuf.at[pl.ds(0, STRIPE % ZR)],
                            acc.at[pl.ds(sid * STRIPE + STRIPE // ZR * ZR,
                                         STRIPE % ZR)])
            plsc.subcore_barrier()

            for u in range(4):
                pltpu.async_copy(y_hbm.at[srcb.at[u]], rows[u], gsem[u])

            def body(t, carry, y_hbm=y_hbm):
                k0 = 4 * t
                for u in range(4):
                    pltpu.make_async_copy(y_hbm.at[srcb.at[k0 + u]], rows[u],
                                          gsem[u]).wait()
                    pltpu.async_copy(rows[u], acc.at[dstb.at[k0 + u]],
                                     ssem[u], add=True)
                for u in range(4):
                    pltpu.make_async_copy(rows[u], acc.at[dstb.at[k0 + u]],
                                          ssem[u]).wait()

                    @pl.when(k0 + u + 4 < KCH)
                    def _(u=u, k0=k0, y_hbm=y_hbm):
                        pltpu.async_copy(y_hbm.at[srcb.at[k0 + u + 4]],
                                         rows[u], gsem[u])
                return carry

            lax.fori_loop(0, KCH // 4, body, 0)
            plsc.subcore_barrier()

            if split:
                pltpu.sync_copy(
                    acc.at[pl.ds(sid * STRIPE, STRIPE)],
                    outs[g].at[pl.ds(sid * STRIPE, STRIPE),
                               pl.ds(cid * F, F)])
            else:
                pltpu.sync_copy(
                    acc.at[pl.ds(sid * STRIPE, STRIPE)],
                    out_hbm.at[pl.ds(sid * STRIPE, STRIPE),
                               pl.ds((g * 2 + cid) * F, F)])

    return k


def _dinv_scale_call(parts, x_i, x_j, W1):
    def body(p_ref, xi_ref, xj_ref, w_ref, d_ref, yi_ref, yj_ref):
        di = lax.rsqrt(1.0 + p_ref[:, 0:1] + p_ref[:, 8:9])
        dj = lax.rsqrt(1.0 + p_ref[:, 16:17] + p_ref[:, 24:25])
        d_ref[:, 0:1] = di
        d_ref[:, 1:2] = dj
        for d, x_ref, y_ref in ((di, xi_ref, yi_ref), (dj, xj_ref, yj_ref)):
            y_ref[0:N] = d[0:N] * jnp.dot(
                x_ref[...], w_ref[...], preferred_element_type=jnp.float32)
            y_ref[N:NP] = jnp.zeros((NP - N, F1), jnp.float32)

    return pl.pallas_call(
        body,
        out_shape=(jax.ShapeDtypeStruct((NP, 2), jnp.float32),
                   jax.ShapeDtypeStruct((NP, F1), jnp.float32),
                   jax.ShapeDtypeStruct((NP, F1), jnp.float32)))(
                       parts, x_i, x_j, W1)


def _make_mid_call(F, Fn, split_a):
    def body(*refs):
        if split_a:
            ai_ref, aj_ref, yi_ref, yj_ref, d_ref, b_ref, w_ref, oi_ref, \
                oj_ref = refs
            slabs = ((ai_ref[:, 0:F], ai_ref[:, F:2 * F]),
                     (aj_ref[:, 0:F], aj_ref[:, F:2 * F]))
        else:
            a_ref, yi_ref, yj_ref, d_ref, b_ref, w_ref, oi_ref, oj_ref = refs
            slabs = ((a_ref[:, 0:F], a_ref[:, F:2 * F]),
                     (a_ref[:, 2 * F:3 * F], a_ref[:, 3 * F:4 * F]))
        for g, (y_ref, o_ref) in enumerate(((yi_ref, oi_ref),
                                            (yj_ref, oj_ref))):
            s = slabs[g][0] + slabs[g][1] + y_ref[...]
            d = d_ref[:, g:g + 1]
            z = jnp.maximum(d * s + b_ref[...], 0.0)
            o_ref[0:N] = d[0:N] * jnp.dot(
                z[0:N], w_ref[...], preferred_element_type=jnp.float32)
            o_ref[N:NP] = jnp.zeros((NP - N, Fn), jnp.float32)

    def call(acc, y_i, y_j, dinv, b, W):
        accs = tuple(acc) if split_a else (acc,)
        return pl.pallas_call(
            body, out_shape=(jax.ShapeDtypeStruct((NP, Fn), jnp.float32),
                             jax.ShapeDtypeStruct((NP, Fn), jnp.float32)))(
                *accs, y_i, y_j, dinv, b, W)

    return call


def _att_head_call(acc, y_i, y_j, dinv, b3r, W_att, W_ntn, V_ntn_t, b_ntn,
                   W_m0, b_m0, W_m1, b_m1, W_m2, b_m2, W_m3, b_m3, W_s, b_s):
    def body(a_ref, yi_ref, yj_ref, d_ref, b_ref, w_ref, wn_ref, vt_ref,
             bn_ref, w0_ref, c0_ref, w1_ref, c1_ref, w2_ref, c2_ref, w3_ref,
             c3_ref, ws_ref, cs_ref, o_ref):
        hs = []
        for g, y_ref in enumerate((yi_ref, yj_ref)):
            s_g = (a_ref[:, 2 * g * F3:(2 * g + 1) * F3]
                   + a_ref[:, (2 * g + 1) * F3:(2 * g + 2) * F3]
                   + y_ref[...])
            enc = d_ref[:, g:g + 1] * s_g + b_ref[...]
            encn = enc[0:N]
            m = jnp.sum(encn, axis=0, keepdims=True) * (1.0 / N)
            c = jnp.tanh(jnp.dot(m, w_ref[...],
                                 preferred_element_type=jnp.float32))
            s = jax.nn.sigmoid(jnp.sum(encn * c, axis=1, keepdims=True))
            hs.append(jnp.sum(encn * s, axis=0, keepdims=True))
        hiv, hjv = hs
        ks = lax.broadcasted_iota(jnp.int32, (1, NTN_SLICES), 1)
        bil = jnp.zeros((1, NTN_SLICES), jnp.float32)
        for k in range(NTN_SLICES):
            wk = wn_ref[k]
            val = jnp.sum(jnp.dot(hiv, wk, preferred_element_type=jnp.float32)
                          * hjv)
            bil = bil + jnp.where(ks == k, val, 0.0)
        cat = jnp.concatenate([hiv, hjv], axis=1)
        lin = jnp.dot(cat, vt_ref[...], preferred_element_type=jnp.float32)
        inter = jnp.tanh(bil + lin + bn_ref[...])
        sim = jax.nn.sigmoid(jnp.sum(hiv * hjv))
        binx = jnp.clip((sim * HIST_BINS).astype(jnp.int32), 0, HIST_BINS - 1)
        hist = jnp.where(ks == binx, 1.0, 0.0)
        feat = jnp.concatenate([inter, hist], axis=1)
        for w_ref, c_ref in ((w0_ref, c0_ref), (w1_ref, c1_ref),
                             (w2_ref, c2_ref), (w3_ref, c3_ref)):
            feat = jnp.maximum(
                jnp.dot(feat, w_ref[...], preferred_element_type=jnp.float32)
                + c_ref[...], 0.0)
        o_ref[...] = jnp.dot(feat, ws_ref[...],
                             preferred_element_type=jnp.float32) + cs_ref[...]

    return pl.pallas_call(
        body, out_shape=jax.ShapeDtypeStruct((1, 1), jnp.float32))(
            acc, y_i, y_j, dinv, b3r, W_att, W_ntn, V_ntn_t, b_ntn,
            W_m0, b_m0, W_m1, b_m1, W_m2, b_m2, W_m3, b_m3, W_s, b_s)


_deg_kernel = _make_degree_kernel()
_seg64 = _make_segsum_kernel(F1)
_seg32 = _make_segsum_kernel(F2)
_seg16 = _make_segsum_kernel(F3)
_mid_64_32 = _make_mid_call(F1, F2, split_a=True)
_mid_32_16 = _make_mid_call(F2, F3, split_a=False)


def _pad_edges(e):
    fill = N + jnp.arange(EP - E, dtype=jnp.int32) % (NP - N)
    return jnp.concatenate([e, fill])


def kernel(x_i, x_j, edge_index_i, edge_index_j, W1, b1, W2, b2, W3, b3,
           W_att, W_ntn, V_ntn, b_ntn, W_m0, b_m0, W_m1, b_m1, W_m2, b_m2,
           W_m3, b_m3, W_s, b_s):
    src = jnp.concatenate([_pad_edges(edge_index_i[0]),
                           _pad_edges(edge_index_j[0])]).reshape(ER, CH)
    dst = jnp.concatenate([_pad_edges(edge_index_i[1]),
                           _pad_edges(edge_index_j[1])]).reshape(ER, CH)

    deg_parts = _deg_kernel(dst)
    dinv, y1_i, y1_j = _dinv_scale_call(deg_parts, x_i, x_j, W1)

    a1 = _seg64(y1_i, y1_j, src, dst)
    y2_i, y2_j = _mid_64_32(a1, y1_i, y1_j, dinv, b1.reshape(1, F1), W2)
    a2 = _seg32(y2_i, y2_j, src, dst)
    y3_i, y3_j = _mid_32_16(a2, y2_i, y2_j, dinv, b2.reshape(1, F2), W3)
    a3 = _seg16(y3_i, y3_j, src, dst)

    out = _att_head_call(a3, y3_i, y3_j, dinv, b3.reshape(1, F3), W_att,
                         W_ntn, V_ntn.T, b_ntn.reshape(1, NTN_SLICES),
                         W_m0, b_m0.reshape(1, 32), W_m1, b_m1.reshape(1, 16),
                         W_m2, b_m2.reshape(1, 8), W_m3, b_m3.reshape(1, 4),
                         W_s, b_s.reshape(1, 1))
    return out.reshape(1)


# speedup vs baseline: 44.0911x; 1.0251x over previous
import functools

import jax
import jax.numpy as jnp
from jax import lax
from jax.experimental import pallas as pl
from jax.experimental.pallas import tpu as pltpu
from jax.experimental.pallas import tpu_sc as plsc

N = 10000
NP = 10112
NT = 2 * NP
E = 320000
D_IN = 128
F1, F2, F3 = 64, 32, 16
NTN_SLICES = 16
HIST_BINS = 16

NWORK = 32
CH = 128
KCH = 80
EC = E // CH
PADC = NWORK * KCH - EC
REAL31 = EC - 31 * KCH
STRIPE = NP // 16
ZR = 120


def _make_degree_kernel():
    DW = 8
    mesh = plsc.VectorSubcoreMesh(core_axis_name="c", subcore_axis_name="s")

    @functools.partial(
        pl.kernel,
        out_type=jax.ShapeDtypeStruct((NP, 4 * DW), jnp.float32),
        mesh=mesh,
        compiler_params=pltpu.CompilerParams(use_tc_tiling_on_sc=False),
        scratch_types=[
            pltpu.VMEM_SHARED((NP, DW), jnp.float32),
            pltpu.VMEM((KCH, CH), jnp.int32),
            pltpu.VMEM((CH, DW), jnp.float32),
            pltpu.VMEM((STRIPE, DW), jnp.float32),
            pltpu.SemaphoreType.DMA,
        ],
    )
    def k(ei_hbm, ej_hbm, pad_hbm, out_hbm, acc, db, ones, zbuf, sem):
        cid = lax.axis_index("c")
        sid = lax.axis_index("s")
        wid = cid * 16 + sid

        def fill_ones(r, carry):
            ones[r, pl.ds(0, DW)] = jnp.ones((DW,), jnp.float32)
            return carry

        def fill_zero(r, carry):
            zbuf[r, pl.ds(0, DW)] = jnp.zeros((DW,), jnp.float32)
            return carry

        lax.fori_loop(0, CH, fill_ones, 0)
        lax.fori_loop(0, STRIPE, fill_zero, 0)

        for g, e_hbm in enumerate((ei_hbm, ej_hbm)):
            @pl.when(wid < 31)
            def _(e_hbm=e_hbm):
                pltpu.sync_copy(e_hbm.at[1, pl.ds(wid * KCH, KCH)], db)

            @pl.when(wid == 31)
            def _(e_hbm=e_hbm):
                pltpu.sync_copy(e_hbm.at[1, pl.ds(31 * KCH, REAL31)],
                                db.at[pl.ds(0, REAL31)])
                pltpu.sync_copy(pad_hbm, db.at[pl.ds(REAL31, PADC)])

            pltpu.sync_copy(zbuf, acc.at[pl.ds(sid * STRIPE, STRIPE)])
            plsc.subcore_barrier()

            def body(t, carry):
                for u in range(4):
                    pltpu.async_copy(ones, acc.at[db.at[t * 4 + u]],
                                     sem, add=True)
                for u in range(4):
                    pltpu.make_async_copy(
                        ones, acc.at[db.at[t * 4 + u]], sem).wait()
                return carry

            lax.fori_loop(0, KCH // 4, body, 0)
            plsc.subcore_barrier()

            pltpu.sync_copy(
                acc.at[pl.ds(sid * STRIPE, STRIPE)],
                out_hbm.at[pl.ds(sid * STRIPE, STRIPE),
                           pl.ds((g * 2 + cid) * DW, DW)])

    return k


def _make_segsum_kernel(F):
    split = F == 64
    if split:
        out_type = (jax.ShapeDtypeStruct((NP, 2 * F), jnp.float32),
                    jax.ShapeDtypeStruct((NP, 2 * F), jnp.float32))
    else:
        out_type = jax.ShapeDtypeStruct((NP, 4 * F), jnp.float32)
    mesh = plsc.VectorSubcoreMesh(core_axis_name="c", subcore_axis_name="s")

    @functools.partial(
        pl.kernel,
        out_type=out_type,
        mesh=mesh,
        compiler_params=pltpu.CompilerParams(use_tc_tiling_on_sc=False),
        scratch_types=[
            pltpu.VMEM_SHARED((NP, F), jnp.float32),
            pltpu.VMEM((KCH, CH), jnp.int32),
            pltpu.VMEM((KCH, CH), jnp.int32),
            [pltpu.VMEM((CH, F), jnp.float32) for _ in range(4)],
            pltpu.VMEM((ZR, F), jnp.float32),
            [pltpu.SemaphoreType.DMA for _ in range(4)],
            [pltpu.SemaphoreType.DMA for _ in range(4)],
        ],
    )
    def k(yi_hbm, yj_hbm, ei_hbm, ej_hbm, pad_hbm, *out_scratch):
        if split:
            (outi_hbm, outj_hbm, acc, srcb, dstb, rows, zbuf, gsem,
             ssem) = out_scratch
            outs = (outi_hbm, outj_hbm)
        else:
            out_hbm, acc, srcb, dstb, rows, zbuf, gsem, ssem = out_scratch
        cid = lax.axis_index("c")
        sid = lax.axis_index("s")
        wid = cid * 16 + sid

        def fill_zero(r, carry):
            for c in range(F // 16):
                zbuf[r, pl.ds(c * 16, 16)] = jnp.zeros((16,), jnp.float32)
            return carry

        lax.fori_loop(0, ZR, fill_zero, 0)

        for g, (y_hbm, e_hbm) in enumerate(((yi_hbm, ei_hbm),
                                            (yj_hbm, ej_hbm))):
            @pl.when(wid < 31)
            def _(e_hbm=e_hbm):
                pltpu.sync_copy(e_hbm.at[0, pl.ds(wid * KCH, KCH)], srcb)
                pltpu.sync_copy(e_hbm.at[1, pl.ds(wid * KCH, KCH)], dstb)

            @pl.when(wid == 31)
            def _(e_hbm=e_hbm):
                pltpu.sync_copy(e_hbm.at[0, pl.ds(31 * KCH, REAL31)],
                                srcb.at[pl.ds(0, REAL31)])
                pltpu.sync_copy(pad_hbm, srcb.at[pl.ds(REAL31, PADC)])
                pltpu.sync_copy(e_hbm.at[1, pl.ds(31 * KCH, REAL31)],
                                dstb.at[pl.ds(0, REAL31)])
                pltpu.sync_copy(pad_hbm, dstb.at[pl.ds(REAL31, PADC)])

            for z in range(STRIPE // ZR):
                pltpu.sync_copy(zbuf,
                                acc.at[pl.ds(sid * STRIPE + z * ZR, ZR)])
            pltpu.sync_copy(zbuf.at[pl.ds(0, STRIPE % ZR)],
                            acc.at[pl.ds(sid * STRIPE + STRIPE // ZR * ZR,
                                         STRIPE % ZR)])
            plsc.subcore_barrier()

            for u in range(4):
                pltpu.async_copy(y_hbm.at[srcb.at[u]], rows[u], gsem[u])

            def body(t, carry, y_hbm=y_hbm):
                k0 = 4 * t
                for u in range(4):
                    pltpu.make_async_copy(y_hbm.at[srcb.at[k0 + u]], rows[u],
                                          gsem[u]).wait()
                    pltpu.async_copy(rows[u], acc.at[dstb.at[k0 + u]],
                                     ssem[u], add=True)
                for u in range(4):
                    pltpu.make_async_copy(rows[u], acc.at[dstb.at[k0 + u]],
                                          ssem[u]).wait()

                    @pl.when(k0 + u + 4 < KCH)
                    def _(u=u, k0=k0, y_hbm=y_hbm):
                        pltpu.async_copy(y_hbm.at[srcb.at[k0 + u + 4]],
                                         rows[u], gsem[u])
                return carry

            lax.fori_loop(0, KCH // 4, body, 0)
            plsc.subcore_barrier()

            if split:
                pltpu.sync_copy(
                    acc.at[pl.ds(sid * STRIPE, STRIPE)],
                    outs[g].at[pl.ds(sid * STRIPE, STRIPE),
                               pl.ds(cid * F, F)])
            else:
                pltpu.sync_copy(
                    acc.at[pl.ds(sid * STRIPE, STRIPE)],
                    out_hbm.at[pl.ds(sid * STRIPE, STRIPE),
                               pl.ds((g * 2 + cid) * F, F)])

    return k


def _dinv_scale_call(parts, x_i, x_j, W1):
    def body(p_ref, xi_ref, xj_ref, w_ref, d_ref, yi_ref, yj_ref):
        di = lax.rsqrt(1.0 + p_ref[:, 0:1] + p_ref[:, 8:9])
        dj = lax.rsqrt(1.0 + p_ref[:, 16:17] + p_ref[:, 24:25])
        d_ref[:, 0:1] = di
        d_ref[:, 1:2] = dj
        for d, x_ref, y_ref in ((di, xi_ref, yi_ref), (dj, xj_ref, yj_ref)):
            y_ref[0:N] = d[0:N] * jnp.dot(
                x_ref[...], w_ref[...], preferred_element_type=jnp.float32)
            y_ref[N:NP] = jnp.zeros((NP - N, F1), jnp.float32)

    return pl.pallas_call(
        body,
        out_shape=(jax.ShapeDtypeStruct((NP, 2), jnp.float32),
                   jax.ShapeDtypeStruct((NP, F1), jnp.float32),
                   jax.ShapeDtypeStruct((NP, F1), jnp.float32)))(
                       parts, x_i, x_j, W1)


def _make_mid_call(F, Fn, split_a):
    def body(*refs):
        if split_a:
            ai_ref, aj_ref, yi_ref, yj_ref, d_ref, b_ref, w_ref, oi_ref, \
                oj_ref = refs
            slabs = ((ai_ref[:, 0:F], ai_ref[:, F:2 * F]),
                     (aj_ref[:, 0:F], aj_ref[:, F:2 * F]))
        else:
            a_ref, yi_ref, yj_ref, d_ref, b_ref, w_ref, oi_ref, oj_ref = refs
            slabs = ((a_ref[:, 0:F], a_ref[:, F:2 * F]),
                     (a_ref[:, 2 * F:3 * F], a_ref[:, 3 * F:4 * F]))
        for g, (y_ref, o_ref) in enumerate(((yi_ref, oi_ref),
                                            (yj_ref, oj_ref))):
            s = slabs[g][0] + slabs[g][1] + y_ref[...]
            d = d_ref[:, g:g + 1]
            z = jnp.maximum(d * s + b_ref[...], 0.0)
            o_ref[0:N] = d[0:N] * jnp.dot(
                z[0:N], w_ref[...], preferred_element_type=jnp.float32)
            o_ref[N:NP] = jnp.zeros((NP - N, Fn), jnp.float32)

    def call(acc, y_i, y_j, dinv, b, W):
        accs = tuple(acc) if split_a else (acc,)
        return pl.pallas_call(
            body, out_shape=(jax.ShapeDtypeStruct((NP, Fn), jnp.float32),
                             jax.ShapeDtypeStruct((NP, Fn), jnp.float32)))(
                *accs, y_i, y_j, dinv, b, W)

    return call


def _att_head_call(acc, y_i, y_j, dinv, b3r, W_att, W_ntn, V_ntn_t, b_ntn,
                   W_m0, b_m0, W_m1, b_m1, W_m2, b_m2, W_m3, b_m3, W_s, b_s):
    def body(a_ref, yi_ref, yj_ref, d_ref, b_ref, w_ref, wn_ref, vt_ref,
             bn_ref, w0_ref, c0_ref, w1_ref, c1_ref, w2_ref, c2_ref, w3_ref,
             c3_ref, ws_ref, cs_ref, o_ref):
        hs = []
        for g, y_ref in enumerate((yi_ref, yj_ref)):
            s_g = (a_ref[:, 2 * g * F3:(2 * g + 1) * F3]
                   + a_ref[:, (2 * g + 1) * F3:(2 * g + 2) * F3]
                   + y_ref[...])
            enc = d_ref[:, g:g + 1] * s_g + b_ref[...]
            encn = enc[0:N]
            m = jnp.sum(encn, axis=0, keepdims=True) * (1.0 / N)
            c = jnp.tanh(jnp.dot(m, w_ref[...],
                                 preferred_element_type=jnp.float32))
            s = jax.nn.sigmoid(jnp.sum(encn * c, axis=1, keepdims=True))
            hs.append(jnp.sum(encn * s, axis=0, keepdims=True))
        hiv, hjv = hs
        ks = lax.broadcasted_iota(jnp.int32, (1, NTN_SLICES), 1)
        bil = jnp.zeros((1, NTN_SLICES), jnp.float32)
        for k in range(NTN_SLICES):
            wk = wn_ref[k]
            val = jnp.sum(jnp.dot(hiv, wk, preferred_element_type=jnp.float32)
                          * hjv)
            bil = bil + jnp.where(ks == k, val, 0.0)
        cat = jnp.concatenate([hiv, hjv], axis=1)
        lin = jnp.dot(cat, vt_ref[...], preferred_element_type=jnp.float32)
        inter = jnp.tanh(bil + lin + bn_ref[...])
        sim = jax.nn.sigmoid(jnp.sum(hiv * hjv))
        binx = jnp.clip((sim * HIST_BINS).astype(jnp.int32), 0, HIST_BINS - 1)
        hist = jnp.where(ks == binx, 1.0, 0.0)
        feat = jnp.concatenate([inter, hist], axis=1)
        for w_ref, c_ref in ((w0_ref, c0_ref), (w1_ref, c1_ref),
                             (w2_ref, c2_ref), (w3_ref, c3_ref)):
            feat = jnp.maximum(
                jnp.dot(feat, w_ref[...], preferred_element_type=jnp.float32)
                + c_ref[...], 0.0)
        o_ref[...] = jnp.dot(feat, ws_ref[...],
                             preferred_element_type=jnp.float32) + cs_ref[...]

    return pl.pallas_call(
        body, out_shape=jax.ShapeDtypeStruct((1, 1), jnp.float32))(
            acc, y_i, y_j, dinv, b3r, W_att, W_ntn, V_ntn_t, b_ntn,
            W_m0, b_m0, W_m1, b_m1, W_m2, b_m2, W_m3, b_m3, W_s, b_s)


_deg_kernel = _make_degree_kernel()
_seg64 = _make_segsum_kernel(F1)
_seg32 = _make_segsum_kernel(F2)
_seg16 = _make_segsum_kernel(F3)
_mid_64_32 = _make_mid_call(F1, F2, split_a=True)
_mid_32_16 = _make_mid_call(F2, F3, split_a=False)


def kernel(x_i, x_j, edge_index_i, edge_index_j, W1, b1, W2, b2, W3, b3,
           W_att, W_ntn, V_ntn, b_ntn, W_m0, b_m0, W_m1, b_m1, W_m2, b_m2,
           W_m3, b_m3, W_s, b_s):
    ei = edge_index_i.reshape(2, EC, CH)
    ej = edge_index_j.reshape(2, EC, CH)
    pad = (N + jnp.arange(PADC * CH, dtype=jnp.int32)
           % (NP - N)).reshape(PADC, CH)

    deg_parts = _deg_kernel(ei, ej, pad)
    dinv, y1_i, y1_j = _dinv_scale_call(deg_parts, x_i, x_j, W1)

    a1 = _seg64(y1_i, y1_j, ei, ej, pad)
    y2_i, y2_j = _mid_64_32(a1, y1_i, y1_j, dinv, b1.reshape(1, F1), W2)
    a2 = _seg32(y2_i, y2_j, ei, ej, pad)
    y3_i, y3_j = _mid_32_16(a2, y2_i, y2_j, dinv, b2.reshape(1, F2), W3)
    a3 = _seg16(y3_i, y3_j, ei, ej, pad)

    out = _att_head_call(a3, y3_i, y3_j, dinv, b3.reshape(1, F3), W_att,
                         W_ntn, V_ntn.T, b_ntn.reshape(1, NTN_SLICES),
                         W_m0, b_m0.reshape(1, 32), W_m1, b_m1.reshape(1, 16),
                         W_m2, b_m2.reshape(1, 8), W_m3, b_m3.reshape(1, 4),
                         W_s, b_s.reshape(1, 1))
    return out.reshape(1)


# speedup vs baseline: 44.1845x; 1.0021x over previous
import functools

import jax
import jax.numpy as jnp
from jax import lax
from jax.experimental import pallas as pl
from jax.experimental.pallas import tpu as pltpu
from jax.experimental.pallas import tpu_sc as plsc

N = 10000
NP = 10112
NT = 2 * NP
E = 320000
D_IN = 128
F1, F2, F3 = 64, 32, 16
NTN_SLICES = 16
HIST_BINS = 16

NWORK = 32
CH = 128
KCH = 80
EC = E // CH
PADC = NWORK * KCH - EC
REAL31 = EC - 31 * KCH
STRIPE = NP // 16
ZR = 120


def _make_degree_kernel():
    DW = 8
    mesh = plsc.VectorSubcoreMesh(core_axis_name="c", subcore_axis_name="s")

    @functools.partial(
        pl.kernel,
        out_type=jax.ShapeDtypeStruct((NP, 4 * DW), jnp.float32),
        mesh=mesh,
        compiler_params=pltpu.CompilerParams(use_tc_tiling_on_sc=False),
        scratch_types=[
            pltpu.VMEM_SHARED((NP, DW), jnp.float32),
            pltpu.VMEM((KCH, CH), jnp.int32),
            pltpu.VMEM((CH, DW), jnp.float32),
            pltpu.VMEM((STRIPE, DW), jnp.float32),
            pltpu.SemaphoreType.DMA,
        ],
    )
    def k(ei_hbm, ej_hbm, pad_hbm, out_hbm, acc, db, ones, zbuf, sem):
        cid = lax.axis_index("c")
        sid = lax.axis_index("s")
        wid = cid * 16 + sid

        def fill_ones(r, carry):
            ones[r, pl.ds(0, DW)] = jnp.ones((DW,), jnp.float32)
            return carry

        def fill_zero(r, carry):
            zbuf[r, pl.ds(0, DW)] = jnp.zeros((DW,), jnp.float32)
            return carry

        lax.fori_loop(0, CH, fill_ones, 0)
        lax.fori_loop(0, STRIPE, fill_zero, 0)

        for g, e_hbm in enumerate((ei_hbm, ej_hbm)):
            @pl.when(wid < 31)
            def _(e_hbm=e_hbm):
                pltpu.sync_copy(e_hbm.at[1, pl.ds(wid * KCH, KCH)], db)

            @pl.when(wid == 31)
            def _(e_hbm=e_hbm):
                pltpu.sync_copy(e_hbm.at[1, pl.ds(31 * KCH, REAL31)],
                                db.at[pl.ds(0, REAL31)])
                pltpu.sync_copy(pad_hbm, db.at[pl.ds(REAL31, PADC)])

            pltpu.sync_copy(zbuf, acc.at[pl.ds(sid * STRIPE, STRIPE)])
            plsc.subcore_barrier()

            def body(t, carry):
                for u in range(4):
                    pltpu.async_copy(ones, acc.at[db.at[t * 4 + u]],
                                     sem, add=True)
                for u in range(4):
                    pltpu.make_async_copy(
                        ones, acc.at[db.at[t * 4 + u]], sem).wait()
                return carry

            lax.fori_loop(0, KCH // 4, body, 0)
            plsc.subcore_barrier()

            pltpu.sync_copy(
                acc.at[pl.ds(sid * STRIPE, STRIPE)],
                out_hbm.at[pl.ds(sid * STRIPE, STRIPE),
                           pl.ds((g * 2 + cid) * DW, DW)])

    return k


def _make_segsum_kernel(F):
    split = F == 64
    if split:
        out_type = (jax.ShapeDtypeStruct((NP, 2 * F), jnp.float32),
                    jax.ShapeDtypeStruct((NP, 2 * F), jnp.float32))
    else:
        out_type = jax.ShapeDtypeStruct((NP, 4 * F), jnp.float32)
    mesh = plsc.VectorSubcoreMesh(core_axis_name="c", subcore_axis_name="s")

    @functools.partial(
        pl.kernel,
        out_type=out_type,
        mesh=mesh,
        compiler_params=pltpu.CompilerParams(use_tc_tiling_on_sc=False),
        scratch_types=[
            pltpu.VMEM_SHARED((NP, F), jnp.float32),
            pltpu.VMEM((KCH, CH), jnp.int32),
            pltpu.VMEM((KCH, CH), jnp.int32),
            [pltpu.VMEM((CH, F), jnp.float32) for _ in range(4)],
            pltpu.VMEM((ZR, F), jnp.float32),
            [pltpu.SemaphoreType.DMA for _ in range(4)],
            [pltpu.SemaphoreType.DMA for _ in range(4)],
        ],
    )
    def k(yi_hbm, yj_hbm, ei_hbm, ej_hbm, pad_hbm, *out_scratch):
        if split:
            (outi_hbm, outj_hbm, acc, srcb, dstb, rows, zbuf, gsem,
             ssem) = out_scratch
            outs = (outi_hbm, outj_hbm)
        else:
            out_hbm, acc, srcb, dstb, rows, zbuf, gsem, ssem = out_scratch
        cid = lax.axis_index("c")
        sid = lax.axis_index("s")
        wid = cid * 16 + sid

        def fill_zero(r, carry):
            for c in range(F // 16):
                zbuf[r, pl.ds(c * 16, 16)] = jnp.zeros((16,), jnp.float32)
            return carry

        lax.fori_loop(0, ZR, fill_zero, 0)

        for g, (y_hbm, e_hbm) in enumerate(((yi_hbm, ei_hbm),
                                            (yj_hbm, ej_hbm))):
            @pl.when(wid < 31)
            def _(e_hbm=e_hbm):
                pltpu.sync_copy(e_hbm.at[0, pl.ds(wid * KCH, KCH)], srcb)
                pltpu.sync_copy(e_hbm.at[1, pl.ds(wid * KCH, KCH)], dstb)

            @pl.when(wid == 31)
            def _(e_hbm=e_hbm):
                pltpu.sync_copy(e_hbm.at[0, pl.ds(31 * KCH, REAL31)],
                                srcb.at[pl.ds(0, REAL31)])
                pltpu.sync_copy(pad_hbm, srcb.at[pl.ds(REAL31, PADC)])
                pltpu.sync_copy(e_hbm.at[1, pl.ds(31 * KCH, REAL31)],
                                dstb.at[pl.ds(0, REAL31)])
                pltpu.sync_copy(pad_hbm, dstb.at[pl.ds(REAL31, PADC)])

            for z in range(STRIPE // ZR):
                pltpu.sync_copy(zbuf,
                                acc.at[pl.ds(sid * STRIPE + z * ZR, ZR)])
            pltpu.sync_copy(zbuf.at[pl.ds(0, STRIPE % ZR)],
                            acc.at[pl.ds(sid * STRIPE + STRIPE // ZR * ZR,
                                         STRIPE % ZR)])
            plsc.subcore_barrier()

            for u in range(4):
                pltpu.async_copy(y_hbm.at[srcb.at[u]], rows[u], gsem[u])

            def body(t, carry, y_hbm=y_hbm):
                k0 = 4 * t
                for u in range(4):
                    pltpu.make_async_copy(y_hbm.at[srcb.at[k0 + u]], rows[u],
                                          gsem[u]).wait()
                    pltpu.async_copy(rows[u], acc.at[dstb.at[k0 + u]],
                                     ssem[u], add=True)
                for u in range(4):
                    pltpu.make_async_copy(rows[u], acc.at[dstb.at[k0 + u]],
                                          ssem[u]).wait()

                    @pl.when(k0 + u + 4 < KCH)
                    def _(u=u, k0=k0, y_hbm=y_hbm):
                        pltpu.async_copy(y_hbm.at[srcb.at[k0 + u + 4]],
                                         rows[u], gsem[u])
                return carry

            lax.fori_loop(0, KCH // 4, body, 0)
            plsc.subcore_barrier()

            if split:
                pltpu.sync_copy(
                    acc.at[pl.ds(sid * STRIPE, STRIPE)],
                    outs[g].at[pl.ds(sid * STRIPE, STRIPE),
                               pl.ds(cid * F, F)])
            else:
                pltpu.sync_copy(
                    acc.at[pl.ds(sid * STRIPE, STRIPE)],
                    out_hbm.at[pl.ds(sid * STRIPE, STRIPE),
                               pl.ds((g * 2 + cid) * F, F)])

    return k


def _dinv_scale_call(parts, x_i, x_j, W1):
    def body(p_ref, xi_ref, xj_ref, w_ref, d_ref, yi_ref, yj_ref):
        di = lax.rsqrt(1.0 + p_ref[:, 0:1] + p_ref[:, 8:9])
        dj = lax.rsqrt(1.0 + p_ref[:, 16:17] + p_ref[:, 24:25])
        d_ref[:, 0:1] = di
        d_ref[:, 1:2] = dj
        for d, x_ref, y_ref in ((di, xi_ref, yi_ref), (dj, xj_ref, yj_ref)):
            y_ref[0:N] = d[0:N] * jnp.dot(
                x_ref[...], w_ref[...], preferred_element_type=jnp.float32)
            y_ref[N:NP] = jnp.zeros((NP - N, F1), jnp.float32)

    return pl.pallas_call(
        body,
        out_shape=(jax.ShapeDtypeStruct((NP, 2), jnp.float32),
                   jax.ShapeDtypeStruct((NP, F1), jnp.float32),
                   jax.ShapeDtypeStruct((NP, F1), jnp.float32)))(
                       parts, x_i, x_j, W1)


def _make_mid_call(F, Fn, split_a):
    def body(*refs):
        if split_a:
            ai_ref, aj_ref, yi_ref, yj_ref, d_ref, b_ref, w_ref, oi_ref, \
                oj_ref = refs
            slabs = ((ai_ref[:, 0:F], ai_ref[:, F:2 * F]),
                     (aj_ref[:, 0:F], aj_ref[:, F:2 * F]))
        else:
            a_ref, yi_ref, yj_ref, d_ref, b_ref, w_ref, oi_ref, oj_ref = refs
            slabs = ((a_ref[:, 0:F], a_ref[:, F:2 * F]),
                     (a_ref[:, 2 * F:3 * F], a_ref[:, 3 * F:4 * F]))
        for g, (y_ref, o_ref) in enumerate(((yi_ref, oi_ref),
                                            (yj_ref, oj_ref))):
            s = slabs[g][0] + slabs[g][1] + y_ref[...]
            d = d_ref[:, g:g + 1]
            z = jnp.maximum(d * s + b_ref[...], 0.0)
            o_ref[0:N] = d[0:N] * jnp.dot(
                z[0:N], w_ref[...], preferred_element_type=jnp.float32)
            o_ref[N:NP] = jnp.zeros((NP - N, Fn), jnp.float32)

    def call(acc, y_i, y_j, dinv, b, W):
        accs = tuple(acc) if split_a else (acc,)
        return pl.pallas_call(
            body, out_shape=(jax.ShapeDtypeStruct((NP, Fn), jnp.float32),
                             jax.ShapeDtypeStruct((NP, Fn), jnp.float32)))(
                *accs, y_i, y_j, dinv, b, W)

    return call


def _att_head_call(acc, y_i, y_j, dinv, b3r, W_att, W_ntn, V_ntn_t, b_ntn,
                   W_m0, b_m0, W_m1, b_m1, W_m2, b_m2, W_m3, b_m3, W_s, b_s):
    def body(a_ref, yi_ref, yj_ref, d_ref, b_ref, w_ref, wn_ref, vt_ref,
             bn_ref, w0_ref, c0_ref, w1_ref, c1_ref, w2_ref, c2_ref, w3_ref,
             c3_ref, ws_ref, cs_ref, o_ref):
        hs = []
        for g, y_ref in enumerate((yi_ref, yj_ref)):
            s_g = (a_ref[:, 2 * g * F3:(2 * g + 1) * F3]
                   + a_ref[:, (2 * g + 1) * F3:(2 * g + 2) * F3]
                   + y_ref[...])
            enc = d_ref[:, g:g + 1] * s_g + b_ref[...]
            encn = enc[0:N]
            m = jnp.sum(encn, axis=0, keepdims=True) * (1.0 / N)
            c = jnp.tanh(jnp.dot(m, w_ref[...],
                                 preferred_element_type=jnp.float32))
            s = jax.nn.sigmoid(jnp.sum(encn * c, axis=1, keepdims=True))
            hs.append(jnp.sum(encn * s, axis=0, keepdims=True))
        hiv, hjv = hs
        ks = lax.broadcasted_iota(jnp.int32, (1, NTN_SLICES), 1)
        bil = jnp.zeros((1, NTN_SLICES), jnp.float32)
        for k in range(NTN_SLICES):
            wk = wn_ref[k]
            val = jnp.sum(jnp.dot(hiv, wk, preferred_element_type=jnp.float32)
                          * hjv)
            bil = bil + jnp.where(ks == k, val, 0.0)
        cat = jnp.concatenate([hiv, hjv], axis=1)
        lin = jnp.dot(cat, vt_ref[...], preferred_element_type=jnp.float32)
        inter = jnp.tanh(bil + lin + bn_ref[...])
        sim = jax.nn.sigmoid(jnp.sum(hiv * hjv))
        binx = jnp.clip((sim * HIST_BINS).astype(jnp.int32), 0, HIST_BINS - 1)
        hist = jnp.where(ks == binx, 1.0, 0.0)
        feat = jnp.concatenate([inter, hist], axis=1)
        for w_ref, c_ref in ((w0_ref, c0_ref), (w1_ref, c1_ref),
                             (w2_ref, c2_ref), (w3_ref, c3_ref)):
            feat = jnp.maximum(
                jnp.dot(feat, w_ref[...], preferred_element_type=jnp.float32)
                + c_ref[...], 0.0)
        o_ref[...] = jnp.dot(feat, ws_ref[...],
                             preferred_element_type=jnp.float32) + cs_ref[...]

    return pl.pallas_call(
        body, out_shape=jax.ShapeDtypeStruct((1, 1), jnp.float32))(
            acc, y_i, y_j, dinv, b3r, W_att, W_ntn, V_ntn_t, b_ntn,
            W_m0, b_m0, W_m1, b_m1, W_m2, b_m2, W_m3, b_m3, W_s, b_s)


@functools.cache
def _deg_kernel_inst():
    return _make_degree_kernel()


@functools.cache
def _segsum_inst(F):
    return _make_segsum_kernel(F)


def _deg_kernel(*a):
    return _deg_kernel_inst()(*a)


def _seg64(*a):
    return _segsum_inst(F1)(*a)


def _seg32(*a):
    return _segsum_inst(F2)(*a)


def _seg16(*a):
    return _segsum_inst(F3)(*a)


_mid_64_32 = _make_mid_call(F1, F2, split_a=True)
_mid_32_16 = _make_mid_call(F2, F3, split_a=False)


def kernel(x_i, x_j, edge_index_i, edge_index_j, W1, b1, W2, b2, W3, b3,
           W_att, W_ntn, V_ntn, b_ntn, W_m0, b_m0, W_m1, b_m1, W_m2, b_m2,
           W_m3, b_m3, W_s, b_s):
    ei = edge_index_i.reshape(2, EC, CH)
    ej = edge_index_j.reshape(2, EC, CH)
    pad = (N + jnp.arange(PADC * CH, dtype=jnp.int32)
           % (NP - N)).reshape(PADC, CH)

    deg_parts = _deg_kernel(ei, ej, pad)
    dinv, y1_i, y1_j = _dinv_scale_call(deg_parts, x_i, x_j, W1)

    a1 = _seg64(y1_i, y1_j, ei, ej, pad)
    y2_i, y2_j = _mid_64_32(a1, y1_i, y1_j, dinv, b1.reshape(1, F1), W2)
    a2 = _seg32(y2_i, y2_j, ei, ej, pad)
    y3_i, y3_j = _mid_32_16(a2, y2_i, y2_j, dinv, b2.reshape(1, F2), W3)
    a3 = _seg16(y3_i, y3_j, ei, ej, pad)

    out = _att_head_call(a3, y3_i, y3_j, dinv, b3.reshape(1, F3), W_att,
                         W_ntn, V_ntn.T, b_ntn.reshape(1, NTN_SLICES),
                         W_m0, b_m0.reshape(1, 32), W_m1, b_m1.reshape(1, 16),
                         W_m2, b_m2.reshape(1, 8), W_m3, b_m3.reshape(1, 4),
                         W_s, b_s.reshape(1, 1))
    return out.reshape(1)
